# Initial kernel scaffold; baseline (speedup 1.0000x reference)
import functools

import jax
import jax.numpy as jnp
from jax import lax
from jax.experimental import pallas as pl
from jax.experimental.pallas import tpu as pltpu
from jax.experimental.pallas import tpu_sc as plsc

N = 10000
E = 320000
D = 128
H1 = 16
H2 = 64

NC = 2
NS = 16
NW = NC * NS

NPAD = 10240
RPT = NPAD // NS
EB = 80
TPE = E // NW
NCHUNK = TPE // EB


def _zero_rows(ref, nrows, ncols):
    per_row = ncols // 16
    z = jnp.zeros((16,), jnp.float32)

    def body(t, carry):
        ref[t // per_row, pl.ds((t % per_row) * 16, 16)] = z
        return carry

    lax.fori_loop(0, nrows * per_row, body, 0)


def _deg_body(ei_hbm, out_hbm, dst_v, ones_v, stage_v, acc, sem):
    c = lax.axis_index("c")
    s = lax.axis_index("s")
    row0 = s * RPT

    z = jnp.zeros((16,), jnp.float32)
    o = jnp.ones((16,), jnp.float32)
    for t in range(EB // 16):
        stage_v[pl.ds(t * 16, 16)] = z
        ones_v[pl.ds(t * 16, 16)] = o
    for j in range(RPT // EB):
        pltpu.sync_copy(stage_v, acc.at[pl.ds(row0 + j * EB, EB)])
    plsc.subcore_barrier()

    ebase = (c * NS + s) * TPE

    def body(i, carry):
        pltpu.sync_copy(ei_hbm.at[1, pl.ds(ebase + i * EB, EB)], dst_v)
        pltpu.sync_copy(ones_v, acc.at[dst_v], add=True)
        return carry

    lax.fori_loop(0, NCHUNK, body, 0)
    plsc.subcore_barrier()

    for j in range(RPT // EB):
        pltpu.sync_copy(acc.at[pl.ds(row0 + j * EB, EB)], stage_v)
        pltpu.sync_copy(stage_v, out_hbm.at[c, pl.ds(row0 + j * EB, EB)])


_deg_call = pl.kernel(
    _deg_body,
    out_type=jax.ShapeDtypeStruct((NC, NPAD), jnp.float32),
    mesh=plsc.VectorSubcoreMesh(core_axis_name="c", subcore_axis_name="s"),
    scratch_types=[
        pltpu.VMEM((EB,), jnp.int32),
        pltpu.VMEM((EB,), jnp.float32),
        pltpu.VMEM((EB,), jnp.float32),
        pltpu.VMEM_SHARED((NPAD,), jnp.float32),
        pltpu.SemaphoreType.DMA,
    ],
)


def _make_agg(F):
    def body(g_hbm, ei_hbm, out_hbm, src_v, dst_v, rows_v, acc, sem):
        c = lax.axis_index("c")
        s = lax.axis_index("s")
        row0 = s * RPT

        _zero_rows(rows_v, EB, F)
        for j in range(RPT // EB):
            pltpu.sync_copy(rows_v, acc.at[pl.ds(row0 + j * EB, EB)])
        plsc.subcore_barrier()

        ebase = (c * NS + s) * TPE

        def loop(i, carry):
            b = ebase + i * EB
            pltpu.sync_copy(ei_hbm.at[0, pl.ds(b, EB)], src_v)
            pltpu.sync_copy(ei_hbm.at[1, pl.ds(b, EB)], dst_v)
            pltpu.async_copy(g_hbm.at[src_v], rows_v, sem).wait()
            pltpu.sync_copy(rows_v, acc.at[dst_v], add=True)
            return carry

        lax.fori_loop(0, NCHUNK, loop, 0)
        plsc.subcore_barrier()

        for j in range(RPT // EB):
            pltpu.sync_copy(acc.at[pl.ds(row0 + j * EB, EB)], rows_v)
            pltpu.sync_copy(rows_v, out_hbm.at[c, pl.ds(row0 + j * EB, EB)])

    return pl.kernel(
        body,
        out_type=jax.ShapeDtypeStruct((NC, NPAD, F), jnp.float32),
        mesh=plsc.VectorSubcoreMesh(core_axis_name="c", subcore_axis_name="s"),
        scratch_types=[
            pltpu.VMEM((EB,), jnp.int32),
            pltpu.VMEM((EB,), jnp.int32),
            pltpu.VMEM((EB, F), jnp.float32),
            pltpu.VMEM_SHARED((NPAD, F), jnp.float32),
            pltpu.SemaphoreType.DMA,
        ],
    )


_agg_h1 = _make_agg(H1)
_agg_h2 = _make_agg(H2)


_R = 2048


def _tc_b_body(x_ref, w1_ref, deg_ref, g1_ref, dinv_ref):
    deg = deg_ref[0, :] + deg_ref[1, :] + 1.0
    dinv = lax.rsqrt(deg)[:, None]
    h = jnp.dot(x_ref[...], w1_ref[...], preferred_element_type=jnp.float32)
    g1_ref[...] = h * dinv
    dinv_ref[...] = dinv


def _tc_b(x, w1, deg):
    return pl.pallas_call(
        _tc_b_body,
        grid=(NPAD // _R,),
        in_specs=[
            pl.BlockSpec((_R, D), lambda i: (i, 0)),
            pl.BlockSpec((D, H1), lambda i: (0, 0)),
            pl.BlockSpec((NC, _R), lambda i: (0, i)),
        ],
        out_specs=[
            pl.BlockSpec((_R, H1), lambda i: (i, 0)),
            pl.BlockSpec((_R, 1), lambda i: (i, 0)),
        ],
        out_shape=[
            jax.ShapeDtypeStruct((NPAD, H1), jnp.float32),
            jax.ShapeDtypeStruct((NPAD, 1), jnp.float32),
        ],
    )(x, w1, deg)


def _tc_d_body(agg_ref, g1_ref, dinv_ref, b1_ref, w2_ref, g2_ref):
    dinv = dinv_ref[...]
    tot = agg_ref[0] + agg_ref[1] + g1_ref[...]
    o1 = jnp.maximum(tot * dinv + b1_ref[...], 0.0)
    g2_ref[...] = jnp.dot(o1 * dinv, w2_ref[...],
                          preferred_element_type=jnp.float32)


def _tc_d(agg1, g1, dinv, b1, w2):
    return pl.pallas_call(
        _tc_d_body,
        grid=(NPAD // _R,),
        in_specs=[
            pl.BlockSpec((NC, _R, H1), lambda i: (0, i, 0)),
            pl.BlockSpec((_R, H1), lambda i: (i, 0)),
            pl.BlockSpec((_R, 1), lambda i: (i, 0)),
            pl.BlockSpec((1, H1), lambda i: (0, 0)),
            pl.BlockSpec((H1, H2), lambda i: (0, 0)),
        ],
        out_specs=pl.BlockSpec((_R, H2), lambda i: (i, 0)),
        out_shape=jax.ShapeDtypeStruct((NPAD, H2), jnp.float32),
    )(agg1, g1, dinv, b1, w2)


def _tc_f_body(agg_ref, g2_ref, dinv_ref, b2_ref, wfc_ref, bfc_ref, out_ref):
    dinv = dinv_ref[...]
    tot = agg_ref[0] + agg_ref[1] + g2_ref[...]
    o2 = jnp.maximum(tot * dinv + b2_ref[...], 0.0)
    y = jnp.dot(o2, wfc_ref[...], preferred_element_type=jnp.float32)
    out_ref[...] = jax.nn.sigmoid(y + bfc_ref[0, 0])


def _tc_f(agg2, g2, dinv, b2, wfc, bfc):
    return pl.pallas_call(
        _tc_f_body,
        grid=(NPAD // _R,),
        in_specs=[
            pl.BlockSpec((NC, _R, H2), lambda i: (0, i, 0)),
            pl.BlockSpec((_R, H2), lambda i: (i, 0)),
            pl.BlockSpec((_R, 1), lambda i: (i, 0)),
            pl.BlockSpec((1, H2), lambda i: (0, 0)),
            pl.BlockSpec((H2, 1), lambda i: (0, 0)),
            pl.BlockSpec((1, 1), lambda i: (0, 0), memory_space=pltpu.SMEM),
        ],
        out_specs=pl.BlockSpec((_R, 1), lambda i: (i, 0)),
        out_shape=jax.ShapeDtypeStruct((NPAD, 1), jnp.float32),
    )(agg2, g2, dinv, b2, wfc, bfc)


@jax.jit
def kernel(x, edge_index, W1, b1, W2, b2, Wfc, bfc):
    x_pad = jnp.zeros((NPAD, D), jnp.float32).at[:N].set(x)
    deg = _deg_call(edge_index)
    g1, dinv = _tc_b(x_pad, W1, deg)
    agg1 = _agg_h1(g1, edge_index)
    g2 = _tc_d(agg1, g1, dinv, b1.reshape(1, H1), W2)
    agg2 = _agg_h2(g2, edge_index)
    out = _tc_f(agg2, g2, dinv, b2.reshape(1, H2), Wfc, bfc.reshape(1, 1))
    return out[:N]


# speedup vs baseline: 15.7848x; 15.7848x over previous
import functools

import jax
import jax.numpy as jnp
from jax import lax
from jax.experimental import pallas as pl
from jax.experimental.pallas import tpu as pltpu
from jax.experimental.pallas import tpu_sc as plsc

N = 10000
E = 320000
D = 128
H1 = 16
H2 = 64

NC = 2
NS = 16
NW = NC * NS

NPAD = 10240
RPT = NPAD // NS
EB = 80
TPE = E // NW
NCHUNK = TPE // EB


def _zero_rows(ref, nrows, ncols):
    per_row = ncols // 16
    z = jnp.zeros((16,), jnp.float32)

    def body(t, carry):
        ref[t // per_row, pl.ds((t % per_row) * 16, 16)] = z
        return carry

    lax.fori_loop(0, nrows * per_row, body, 0)


def _deg_body(dst_hbm, out_hbm, dst_v, ones_v, stage_v, acc, sem):
    c = lax.axis_index("c")
    s = lax.axis_index("s")
    row0 = s * RPT

    z = jnp.zeros((16,), jnp.float32)
    o = jnp.ones((16,), jnp.float32)
    for t in range(EB // 16):
        stage_v[pl.ds(t * 16, 16)] = z
        ones_v[pl.ds(t * 16, 16)] = o
    for j in range(RPT // EB):
        pltpu.sync_copy(stage_v, acc.at[pl.ds(row0 + j * EB, EB)])
    plsc.subcore_barrier()

    ebase = (c * NS + s) * TPE

    def body(i, carry):
        pltpu.sync_copy(dst_hbm.at[pl.ds(ebase + i * EB, EB)], dst_v)
        pltpu.sync_copy(ones_v, acc.at[dst_v], add=True)
        return carry

    lax.fori_loop(0, NCHUNK, body, 0)
    plsc.subcore_barrier()

    for j in range(RPT // EB):
        pltpu.sync_copy(acc.at[pl.ds(row0 + j * EB, EB)], stage_v)
        pltpu.sync_copy(stage_v, out_hbm.at[pl.ds(c * NPAD + row0 + j * EB, EB)])


_deg_call = pl.kernel(
    _deg_body,
    out_type=jax.ShapeDtypeStruct((NC * NPAD,), jnp.float32),
    mesh=plsc.VectorSubcoreMesh(core_axis_name="c", subcore_axis_name="s"),
    scratch_types=[
        pltpu.VMEM((EB,), jnp.int32),
        pltpu.VMEM((EB,), jnp.float32),
        pltpu.VMEM((EB,), jnp.float32),
        pltpu.VMEM_SHARED((NPAD,), jnp.float32),
        pltpu.SemaphoreType.DMA,
    ],
)


def _make_agg(F):
    def body(g_hbm, src_hbm, dst_hbm, out_hbm, src_v, dst_v, rows_v, acc, sem):
        c = lax.axis_index("c")
        s = lax.axis_index("s")
        row0 = s * RPT

        _zero_rows(rows_v, EB, F)
        for j in range(RPT // EB):
            pltpu.sync_copy(rows_v, acc.at[pl.ds(row0 + j * EB, EB)])
        plsc.subcore_barrier()

        ebase = (c * NS + s) * TPE

        def loop(i, carry):
            b = ebase + i * EB
            pltpu.sync_copy(src_hbm.at[pl.ds(b, EB)], src_v)
            pltpu.sync_copy(dst_hbm.at[pl.ds(b, EB)], dst_v)
            pltpu.async_copy(g_hbm.at[src_v], rows_v, sem).wait()
            pltpu.sync_copy(rows_v, acc.at[dst_v], add=True)
            return carry

        lax.fori_loop(0, NCHUNK, loop, 0)
        plsc.subcore_barrier()

        for j in range(RPT // EB):
            pltpu.sync_copy(acc.at[pl.ds(row0 + j * EB, EB)], rows_v)
            pltpu.sync_copy(rows_v, out_hbm.at[c, pl.ds(row0 + j * EB, EB)])

    return pl.kernel(
        body,
        out_type=jax.ShapeDtypeStruct((NC, NPAD, F), jnp.float32),
        mesh=plsc.VectorSubcoreMesh(core_axis_name="c", subcore_axis_name="s"),
        compiler_params=pltpu.CompilerParams(use_tc_tiling_on_sc=False),
        scratch_types=[
            pltpu.VMEM((EB,), jnp.int32),
            pltpu.VMEM((EB,), jnp.int32),
            pltpu.VMEM((EB, F), jnp.float32),
            pltpu.VMEM_SHARED((NPAD, F), jnp.float32),
            pltpu.SemaphoreType.DMA,
        ],
    )


_agg_h1 = _make_agg(H1)
_agg_h2 = _make_agg(H2)


_R = 2048


def _tc_b_body(x_ref, w1_ref, deg_ref, g1_ref, dinv_ref):
    deg = deg_ref[0, :] + deg_ref[1, :] + 1.0
    dinv = lax.rsqrt(deg)[:, None]
    h = jnp.dot(x_ref[...], w1_ref[...], preferred_element_type=jnp.float32)
    g1_ref[...] = h * dinv
    dinv_ref[...] = dinv


def _tc_b(x, w1, deg):
    return pl.pallas_call(
        _tc_b_body,
        grid=(NPAD // _R,),
        in_specs=[
            pl.BlockSpec((_R, D), lambda i: (i, 0)),
            pl.BlockSpec((D, H1), lambda i: (0, 0)),
            pl.BlockSpec((NC, _R), lambda i: (0, i)),
        ],
        out_specs=[
            pl.BlockSpec((_R, H1), lambda i: (i, 0)),
            pl.BlockSpec((_R, 1), lambda i: (i, 0)),
        ],
        out_shape=[
            jax.ShapeDtypeStruct((NPAD, H1), jnp.float32),
            jax.ShapeDtypeStruct((NPAD, 1), jnp.float32),
        ],
    )(x, w1, deg)


def _tc_d_body(agg_ref, g1_ref, dinv_ref, b1_ref, w2_ref, g2_ref):
    dinv = dinv_ref[...]
    tot = agg_ref[0] + agg_ref[1] + g1_ref[...]
    o1 = jnp.maximum(tot * dinv + b1_ref[...], 0.0)
    g2_ref[...] = jnp.dot(o1 * dinv, w2_ref[...],
                          preferred_element_type=jnp.float32)


def _tc_d(agg1, g1, dinv, b1, w2):
    return pl.pallas_call(
        _tc_d_body,
        grid=(NPAD // _R,),
        in_specs=[
            pl.BlockSpec((NC, _R, H1), lambda i: (0, i, 0)),
            pl.BlockSpec((_R, H1), lambda i: (i, 0)),
            pl.BlockSpec((_R, 1), lambda i: (i, 0)),
            pl.BlockSpec((1, H1), lambda i: (0, 0)),
            pl.BlockSpec((H1, H2), lambda i: (0, 0)),
        ],
        out_specs=pl.BlockSpec((_R, H2), lambda i: (i, 0)),
        out_shape=jax.ShapeDtypeStruct((NPAD, H2), jnp.float32),
    )(agg1, g1, dinv, b1, w2)


def _tc_f_body(agg_ref, g2_ref, dinv_ref, b2_ref, wfc_ref, bfc_ref, out_ref):
    dinv = dinv_ref[...]
    tot = agg_ref[0] + agg_ref[1] + g2_ref[...]
    o2 = jnp.maximum(tot * dinv + b2_ref[...], 0.0)
    y = jnp.dot(o2, wfc_ref[...], preferred_element_type=jnp.float32)
    out_ref[...] = jax.nn.sigmoid(y + bfc_ref[0, 0])


def _tc_f(agg2, g2, dinv, b2, wfc, bfc):
    return pl.pallas_call(
        _tc_f_body,
        grid=(NPAD // _R,),
        in_specs=[
            pl.BlockSpec((NC, _R, H2), lambda i: (0, i, 0)),
            pl.BlockSpec((_R, H2), lambda i: (i, 0)),
            pl.BlockSpec((_R, 1), lambda i: (i, 0)),
            pl.BlockSpec((1, H2), lambda i: (0, 0)),
            pl.BlockSpec((H2, 1), lambda i: (0, 0)),
            pl.BlockSpec((1, 1), lambda i: (0, 0), memory_space=pltpu.SMEM),
        ],
        out_specs=pl.BlockSpec((_R, 1), lambda i: (i, 0)),
        out_shape=jax.ShapeDtypeStruct((NPAD, 1), jnp.float32),
    )(agg2, g2, dinv, b2, wfc, bfc)


@jax.jit
def kernel(x, edge_index, W1, b1, W2, b2, Wfc, bfc):
    x_pad = jnp.zeros((NPAD, D), jnp.float32).at[:N].set(x)
    src = edge_index[0]
    dst = edge_index[1]
    deg = _deg_call(dst).reshape(NC, NPAD)
    g1, dinv = _tc_b(x_pad, W1, deg)
    agg1 = _agg_h1(g1, src, dst)
    g2 = _tc_d(agg1, g1, dinv, b1.reshape(1, H1), W2)
    agg2 = _agg_h2(g2, src, dst)
    out = _tc_f(agg2, g2, dinv, b2.reshape(1, H2), Wfc, bfc.reshape(1, 1))
    return out[:N]


# speedup vs baseline: 47.6595x; 3.0193x over previous
import functools

import jax
import jax.numpy as jnp
from jax import lax
from jax.experimental import pallas as pl
from jax.experimental.pallas import tpu as pltpu
from jax.experimental.pallas import tpu_sc as plsc

N = 10000
E = 320000
D = 128
H1 = 16
H2 = 64

NC = 2
NS = 16
NW = NC * NS

NPAD = 10240
RPT = NPAD // NS
EB = 80
TPE = E // NW
NCHUNK = TPE // EB
NBUF = 5
NG = NCHUNK // NBUF
WBC = RPT // EB
IDXB = 16


def _zero_rows(ref, nrows, ncols):
    per_row = ncols // 16
    z = jnp.zeros((16,), jnp.float32)

    def body(t, carry):
        ref[t // per_row, pl.ds((t % per_row) * 16, 16)] = z
        return carry

    lax.fori_loop(0, nrows * per_row, body, 0)


def _preload_dst(dst_hbm, dst2d, ebase, sem):
    for k0 in range(0, NCHUNK, IDXB):
        descs = [
            pltpu.async_copy(dst_hbm.at[pl.ds(ebase + i * EB, EB)],
                             dst2d.at[i], sem)
            for i in range(k0, min(k0 + IDXB, NCHUNK))
        ]
        for d in descs:
            d.wait()


def _deg_body(dst_hbm, out_hbm, dst2d, ones_v, stage_v, acc,
              sem_i, ss0, ss1, ss2, ss3, ss4):
    sems = (ss0, ss1, ss2, ss3, ss4)
    c = lax.axis_index("c")
    s = lax.axis_index("s")
    row0 = s * RPT
    ebase = (c * NS + s) * TPE

    _preload_dst(dst_hbm, dst2d, ebase, sem_i)

    z = jnp.zeros((16,), jnp.float32)
    o = jnp.ones((16,), jnp.float32)
    for t in range(EB // 16):
        stage_v[pl.ds(t * 16, 16)] = z
        ones_v[pl.ds(t * 16, 16)] = o
    zd = [
        pltpu.async_copy(stage_v, acc.at[pl.ds(row0 + j * EB, EB)], sem_i)
        for j in range(WBC)
    ]
    for d in zd:
        d.wait()
    plsc.subcore_barrier()

    def grp(t, carry):
        for b in range(NBUF):
            i = t * NBUF + b

            @pl.when(t > 0)
            def _():
                pltpu.make_async_copy(ones_v, acc.at[dst2d.at[i]],
                                      sems[b]).wait()

            pltpu.async_copy(ones_v, acc.at[dst2d.at[i]], sems[b], add=True)
        return carry

    lax.fori_loop(0, NG, grp, 0)
    for b in range(NBUF):
        pltpu.make_async_copy(ones_v, acc.at[dst2d.at[b]], sems[b]).wait()
    plsc.subcore_barrier()

    for j in range(WBC):
        pltpu.sync_copy(acc.at[pl.ds(row0 + j * EB, EB)], stage_v)
        pltpu.sync_copy(stage_v,
                        out_hbm.at[pl.ds(c * NPAD + row0 + j * EB, EB)])


_deg_call = pl.kernel(
    _deg_body,
    out_type=jax.ShapeDtypeStruct((NC * NPAD,), jnp.float32),
    mesh=plsc.VectorSubcoreMesh(core_axis_name="c", subcore_axis_name="s"),
    compiler_params=pltpu.CompilerParams(use_tc_tiling_on_sc=False),
    scratch_types=[
        pltpu.VMEM((NCHUNK, EB), jnp.int32),
        pltpu.VMEM((EB,), jnp.float32),
        pltpu.VMEM((EB,), jnp.float32),
        pltpu.VMEM_SHARED((NPAD,), jnp.float32),
        pltpu.SemaphoreType.DMA,
        pltpu.SemaphoreType.DMA,
        pltpu.SemaphoreType.DMA,
        pltpu.SemaphoreType.DMA,
        pltpu.SemaphoreType.DMA,
        pltpu.SemaphoreType.DMA,
    ],
)


def _make_agg(F):
    def body(g_hbm, src_hbm, dst_hbm, out_hbm,
             src_all, dst2d, r0, r1, r2, r3, r4, acc,
             sem_i, sg0, sg1, sg2, sg3, sg4, ss0, ss1, ss2, ss3, ss4):
        rows = (r0, r1, r2, r3, r4)
        semg = (sg0, sg1, sg2, sg3, sg4)
        sems = (ss0, ss1, ss2, ss3, ss4)
        c = lax.axis_index("c")
        s = lax.axis_index("s")
        row0 = s * RPT
        ebase = (c * NS + s) * TPE

        dsrc = pltpu.async_copy(src_hbm.at[pl.ds(ebase, TPE)], src_all, sg0)
        _preload_dst(dst_hbm, dst2d, ebase, sem_i)
        dsrc.wait()

        _zero_rows(rows[0], EB, F)
        zd = [
            pltpu.async_copy(rows[0], acc.at[pl.ds(row0 + j * EB, EB)], sem_i)
            for j in range(WBC)
        ]
        for d in zd:
            d.wait()
        plsc.subcore_barrier()

        def grp(t, carry):
            for b in range(NBUF):
                i = t * NBUF + b

                @pl.when(t > 0)
                def _():
                    pltpu.make_async_copy(rows[b], acc.at[dst2d.at[i]],
                                          sems[b]).wait()

                pltpu.async_copy(
                    g_hbm.at[src_all.at[pl.ds(i * EB, EB)]], rows[b], semg[b])
            for b in range(NBUF):
                i = t * NBUF + b
                pltpu.make_async_copy(
                    g_hbm.at[src_all.at[pl.ds(i * EB, EB)]], rows[b],
                    semg[b]).wait()
                pltpu.async_copy(rows[b], acc.at[dst2d.at[i]], sems[b],
                                 add=True)
            return carry

        lax.fori_loop(0, NG, grp, 0)
        for b in range(NBUF):
            pltpu.make_async_copy(rows[b], acc.at[dst2d.at[b]],
                                  sems[b]).wait()
        plsc.subcore_barrier()

        for j in range(NBUF):
            pltpu.async_copy(acc.at[pl.ds(row0 + j * EB, EB)], rows[j],
                             semg[j])
        for j in range(NBUF):
            pltpu.make_async_copy(acc.at[pl.ds(row0 + j * EB, EB)], rows[j],
                                  semg[j]).wait()
            pltpu.async_copy(rows[j],
                             out_hbm.at[c, pl.ds(row0 + j * EB, EB)], sems[j])
        for j in range(WBC - NBUF):
            jj = NBUF + j
            pltpu.make_async_copy(
                rows[j], out_hbm.at[c, pl.ds(row0, EB)], sems[j]).wait()
            pltpu.async_copy(acc.at[pl.ds(row0 + jj * EB, EB)], rows[j],
                             semg[j])
            pltpu.make_async_copy(acc.at[pl.ds(row0 + jj * EB, EB)], rows[j],
                                  semg[j]).wait()
            pltpu.async_copy(rows[j],
                             out_hbm.at[c, pl.ds(row0 + jj * EB, EB)],
                             sems[j])
        for j in range(NBUF):
            pltpu.make_async_copy(
                rows[j], out_hbm.at[c, pl.ds(row0, EB)], sems[j]).wait()

    return pl.kernel(
        body,
        out_type=jax.ShapeDtypeStruct((NC, NPAD, F), jnp.float32),
        mesh=plsc.VectorSubcoreMesh(core_axis_name="c", subcore_axis_name="s"),
        compiler_params=pltpu.CompilerParams(use_tc_tiling_on_sc=False),
        scratch_types=[
            pltpu.VMEM((TPE,), jnp.int32),
            pltpu.VMEM((NCHUNK, EB), jnp.int32),
            pltpu.VMEM((EB, F), jnp.float32),
            pltpu.VMEM((EB, F), jnp.float32),
            pltpu.VMEM((EB, F), jnp.float32),
            pltpu.VMEM((EB, F), jnp.float32),
            pltpu.VMEM((EB, F), jnp.float32),
            pltpu.VMEM_SHARED((NPAD, F), jnp.float32),
            pltpu.SemaphoreType.DMA,
            pltpu.SemaphoreType.DMA,
            pltpu.SemaphoreType.DMA,
            pltpu.SemaphoreType.DMA,
            pltpu.SemaphoreType.DMA,
            pltpu.SemaphoreType.DMA,
            pltpu.SemaphoreType.DMA,
            pltpu.SemaphoreType.DMA,
            pltpu.SemaphoreType.DMA,
            pltpu.SemaphoreType.DMA,
            pltpu.SemaphoreType.DMA,
        ],
    )


_agg_h1 = _make_agg(H1)
_agg_h2 = _make_agg(H2)


_R = 2048


def _tc_b_body(x_ref, w1_ref, deg_ref, g1_ref, dinv_ref):
    deg = deg_ref[0, :] + deg_ref[1, :] + 1.0
    dinv = lax.rsqrt(deg)[:, None]
    h = jnp.dot(x_ref[...], w1_ref[...], preferred_element_type=jnp.float32)
    g1_ref[...] = h * dinv
    dinv_ref[...] = dinv


def _tc_b(x, w1, deg):
    return pl.pallas_call(
        _tc_b_body,
        grid=(NPAD // _R,),
        in_specs=[
            pl.BlockSpec((_R, D), lambda i: (i, 0)),
            pl.BlockSpec((D, H1), lambda i: (0, 0)),
            pl.BlockSpec((NC, _R), lambda i: (0, i)),
        ],
        out_specs=[
            pl.BlockSpec((_R, H1), lambda i: (i, 0)),
            pl.BlockSpec((_R, 1), lambda i: (i, 0)),
        ],
        out_shape=[
            jax.ShapeDtypeStruct((NPAD, H1), jnp.float32),
            jax.ShapeDtypeStruct((NPAD, 1), jnp.float32),
        ],
    )(x, w1, deg)


def _tc_d_body(agg_ref, g1_ref, dinv_ref, b1_ref, w2_ref, g2_ref):
    dinv = dinv_ref[...]
    tot = agg_ref[0] + agg_ref[1] + g1_ref[...]
    o1 = jnp.maximum(tot * dinv + b1_ref[...], 0.0)
    g2_ref[...] = jnp.dot(o1 * dinv, w2_ref[...],
                          preferred_element_type=jnp.float32)


def _tc_d(agg1, g1, dinv, b1, w2):
    return pl.pallas_call(
        _tc_d_body,
        grid=(NPAD // _R,),
        in_specs=[
            pl.BlockSpec((NC, _R, H1), lambda i: (0, i, 0)),
            pl.BlockSpec((_R, H1), lambda i: (i, 0)),
            pl.BlockSpec((_R, 1), lambda i: (i, 0)),
            pl.BlockSpec((1, H1), lambda i: (0, 0)),
            pl.BlockSpec((H1, H2), lambda i: (0, 0)),
        ],
        out_specs=pl.BlockSpec((_R, H2), lambda i: (i, 0)),
        out_shape=jax.ShapeDtypeStruct((NPAD, H2), jnp.float32),
    )(agg1, g1, dinv, b1, w2)


def _tc_f_body(agg_ref, g2_ref, dinv_ref, b2_ref, wfc_ref, bfc_ref, out_ref):
    dinv = dinv_ref[...]
    tot = agg_ref[0] + agg_ref[1] + g2_ref[...]
    o2 = jnp.maximum(tot * dinv + b2_ref[...], 0.0)
    y = jnp.dot(o2, wfc_ref[...], preferred_element_type=jnp.float32)
    out_ref[...] = jax.nn.sigmoid(y + bfc_ref[0, 0])


def _tc_f(agg2, g2, dinv, b2, wfc, bfc):
    return pl.pallas_call(
        _tc_f_body,
        grid=(NPAD // _R,),
        in_specs=[
            pl.BlockSpec((NC, _R, H2), lambda i: (0, i, 0)),
            pl.BlockSpec((_R, H2), lambda i: (i, 0)),
            pl.BlockSpec((_R, 1), lambda i: (i, 0)),
            pl.BlockSpec((1, H2), lambda i: (0, 0)),
            pl.BlockSpec((H2, 1), lambda i: (0, 0)),
            pl.BlockSpec((1, 1), lambda i: (0, 0), memory_space=pltpu.SMEM),
        ],
        out_specs=pl.BlockSpec((_R, 1), lambda i: (i, 0)),
        out_shape=jax.ShapeDtypeStruct((NPAD, 1), jnp.float32),
    )(agg2, g2, dinv, b2, wfc, bfc)


@jax.jit
def kernel(x, edge_index, W1, b1, W2, b2, Wfc, bfc):
    x_pad = jnp.zeros((NPAD, D), jnp.float32).at[:N].set(x)
    src = edge_index[0]
    dst = edge_index[1]
    deg = _deg_call(dst).reshape(NC, NPAD)
    g1, dinv = _tc_b(x_pad, W1, deg)
    agg1 = _agg_h1(g1, src, dst)
    g2 = _tc_d(agg1, g1, dinv, b1.reshape(1, H1), W2)
    agg2 = _agg_h2(g2, src, dst)
    out = _tc_f(agg2, g2, dinv, b2.reshape(1, H2), Wfc, bfc.reshape(1, 1))
    return out[:N]


# speedup vs baseline: 50.5602x; 1.0609x over previous
import functools

import jax
import jax.numpy as jnp
from jax import lax
from jax.experimental import pallas as pl
from jax.experimental.pallas import tpu as pltpu
from jax.experimental.pallas import tpu_sc as plsc

N = 10000
E = 320000
D = 128
H1 = 16
H2 = 64

NC = 2
NS = 16
NW = NC * NS

NPAD = 10240
RPT = NPAD // NS
EB = 128
CN = 78
NBUF = 6
NG = CN // NBUF
WBC = RPT // EB
IDXB = 16
XTRA = E - NW * CN * EB


def _zero_rows(ref, nrows, ncols):
    per_row = ncols // 16
    z = jnp.zeros((16,), jnp.float32)

    def body(t, carry):
        ref[t // per_row, pl.ds((t % per_row) * 16, 16)] = z
        return carry

    lax.fori_loop(0, nrows * per_row, body, 0)


def _edge_base(wid):
    return wid * (CN * EB) + jnp.minimum(wid, XTRA // EB) * EB


def _preload_dst(dst_hbm, dst2d, ebase, wid, sem):
    for k0 in range(0, CN, IDXB):
        descs = [
            pltpu.async_copy(dst_hbm.at[pl.ds(ebase + i * EB, EB)],
                             dst2d.at[i], sem)
            for i in range(k0, min(k0 + IDXB, CN))
        ]
        for d in descs:
            d.wait()

    @pl.when(wid < XTRA // EB)
    def _():
        pltpu.async_copy(dst_hbm.at[pl.ds(ebase + CN * EB, EB)],
                         dst2d.at[CN], sem).wait()


def _deg_body(dst_hbm, out_hbm, dst2d, ones_v, stage_v, acc,
              sem_i, ss0, ss1, ss2, ss3, ss4, ss5):
    sems = (ss0, ss1, ss2, ss3, ss4, ss5)
    c = lax.axis_index("c")
    s = lax.axis_index("s")
    row0 = s * RPT
    wid = c * NS + s
    ebase = _edge_base(wid)

    _preload_dst(dst_hbm, dst2d, ebase, wid, sem_i)

    z = jnp.zeros((16,), jnp.float32)
    o = jnp.ones((16,), jnp.float32)
    for t in range(EB // 16):
        stage_v[pl.ds(t * 16, 16)] = z
        ones_v[pl.ds(t * 16, 16)] = o
    zd = [
        pltpu.async_copy(stage_v, acc.at[pl.ds(row0 + j * EB, EB)], sem_i)
        for j in range(WBC)
    ]
    for d in zd:
        d.wait()
    plsc.subcore_barrier()

    def grp(t, carry):
        for b in range(NBUF):
            i = t * NBUF + b

            @pl.when(t > 0)
            def _():
                pltpu.make_async_copy(ones_v, acc.at[dst2d.at[i]],
                                      sems[b]).wait()

            pltpu.async_copy(ones_v, acc.at[dst2d.at[i]], sems[b], add=True)
        return carry

    lax.fori_loop(0, NG, grp, 0)
    for b in range(NBUF):
        pltpu.make_async_copy(ones_v, acc.at[dst2d.at[b]], sems[b]).wait()

    @pl.when(wid < XTRA // EB)
    def _():
        pltpu.async_copy(ones_v, acc.at[dst2d.at[CN]], sems[0], add=True)
        pltpu.make_async_copy(ones_v, acc.at[dst2d.at[CN]], sems[0]).wait()

    plsc.subcore_barrier()

    for j in range(WBC):
        pltpu.sync_copy(acc.at[pl.ds(row0 + j * EB, EB)], stage_v)
        pltpu.sync_copy(stage_v,
                        out_hbm.at[pl.ds(c * NPAD + row0 + j * EB, EB)])


_deg_call = pl.kernel(
    _deg_body,
    out_type=jax.ShapeDtypeStruct((NC * NPAD,), jnp.float32),
    mesh=plsc.VectorSubcoreMesh(core_axis_name="c", subcore_axis_name="s"),
    compiler_params=pltpu.CompilerParams(use_tc_tiling_on_sc=False),
    scratch_types=[
        pltpu.VMEM((CN + 1, EB), jnp.int32),
        pltpu.VMEM((EB,), jnp.float32),
        pltpu.VMEM((EB,), jnp.float32),
        pltpu.VMEM_SHARED((NPAD,), jnp.float32),
        pltpu.SemaphoreType.DMA,
        pltpu.SemaphoreType.DMA,
        pltpu.SemaphoreType.DMA,
        pltpu.SemaphoreType.DMA,
        pltpu.SemaphoreType.DMA,
        pltpu.SemaphoreType.DMA,
        pltpu.SemaphoreType.DMA,
    ],
)


def _make_agg(F):
    def body(g_hbm, src_hbm, dst_hbm, out_hbm,
             src_all, dst2d, r0, r1, r2, r3, r4, r5, acc,
             sem_i, sg0, sg1, sg2, sg3, sg4, sg5,
             ss0, ss1, ss2, ss3, ss4, ss5):
        rows = (r0, r1, r2, r3, r4, r5)
        semg = (sg0, sg1, sg2, sg3, sg4, sg5)
        sems = (ss0, ss1, ss2, ss3, ss4, ss5)
        c = lax.axis_index("c")
        s = lax.axis_index("s")
        row0 = s * RPT
        wid = c * NS + s
        ebase = _edge_base(wid)

        dsrc = pltpu.async_copy(src_hbm.at[pl.ds(ebase, CN * EB)],
                                src_all.at[pl.ds(0, CN * EB)], sg0)
        _preload_dst(dst_hbm, dst2d, ebase, wid, sem_i)

        @pl.when(wid < XTRA // EB)
        def _():
            pltpu.async_copy(src_hbm.at[pl.ds(ebase + CN * EB, EB)],
                             src_all.at[pl.ds(CN * EB, EB)], sg1).wait()

        dsrc.wait()

        _zero_rows(rows[0], EB, F)
        zd = [
            pltpu.async_copy(rows[0], acc.at[pl.ds(row0 + j * EB, EB)], sem_i)
            for j in range(WBC)
        ]
        for d in zd:
            d.wait()
        plsc.subcore_barrier()

        def grp(t, carry):
            for b in range(NBUF):
                i = t * NBUF + b

                @pl.when(t > 0)
                def _():
                    pltpu.make_async_copy(rows[b], acc.at[dst2d.at[i]],
                                          sems[b]).wait()

                pltpu.async_copy(
                    g_hbm.at[src_all.at[pl.ds(i * EB, EB)]], rows[b], semg[b])
            for b in range(NBUF):
                i = t * NBUF + b
                pltpu.make_async_copy(
                    g_hbm.at[src_all.at[pl.ds(i * EB, EB)]], rows[b],
                    semg[b]).wait()
                pltpu.async_copy(rows[b], acc.at[dst2d.at[i]], sems[b],
                                 add=True)
            return carry

        lax.fori_loop(0, NG, grp, 0)
        for b in range(NBUF):
            pltpu.make_async_copy(rows[b], acc.at[dst2d.at[b]],
                                  sems[b]).wait()

        @pl.when(wid < XTRA // EB)
        def _():
            pltpu.async_copy(
                g_hbm.at[src_all.at[pl.ds(CN * EB, EB)]], rows[0], semg[0])
            pltpu.make_async_copy(
                g_hbm.at[src_all.at[pl.ds(CN * EB, EB)]], rows[0],
                semg[0]).wait()
            pltpu.async_copy(rows[0], acc.at[dst2d.at[CN]], sems[0],
                             add=True)
            pltpu.make_async_copy(rows[0], acc.at[dst2d.at[CN]],
                                  sems[0]).wait()

        plsc.subcore_barrier()

        for j in range(WBC):
            pltpu.async_copy(acc.at[pl.ds(row0 + j * EB, EB)], rows[j],
                             semg[j])
        for j in range(WBC):
            pltpu.make_async_copy(acc.at[pl.ds(row0 + j * EB, EB)], rows[j],
                                  semg[j]).wait()
            pltpu.async_copy(rows[j],
                             out_hbm.at[c, pl.ds(row0 + j * EB, EB)], sems[j])
        for j in range(WBC):
            pltpu.make_async_copy(
                rows[j], out_hbm.at[c, pl.ds(row0, EB)], sems[j]).wait()

    return pl.kernel(
        body,
        out_type=jax.ShapeDtypeStruct((NC, NPAD, F), jnp.float32),
        mesh=plsc.VectorSubcoreMesh(core_axis_name="c", subcore_axis_name="s"),
        compiler_params=pltpu.CompilerParams(use_tc_tiling_on_sc=False),
        scratch_types=[
            pltpu.VMEM(((CN + 1) * EB,), jnp.int32),
            pltpu.VMEM((CN + 1, EB), jnp.int32),
            pltpu.VMEM((EB, F), jnp.float32),
            pltpu.VMEM((EB, F), jnp.float32),
            pltpu.VMEM((EB, F), jnp.float32),
            pltpu.VMEM((EB, F), jnp.float32),
            pltpu.VMEM((EB, F), jnp.float32),
            pltpu.VMEM((EB, F), jnp.float32),
            pltpu.VMEM_SHARED((NPAD, F), jnp.float32),
            pltpu.SemaphoreType.DMA,
            pltpu.SemaphoreType.DMA,
            pltpu.SemaphoreType.DMA,
            pltpu.SemaphoreType.DMA,
            pltpu.SemaphoreType.DMA,
            pltpu.SemaphoreType.DMA,
            pltpu.SemaphoreType.DMA,
            pltpu.SemaphoreType.DMA,
            pltpu.SemaphoreType.DMA,
            pltpu.SemaphoreType.DMA,
            pltpu.SemaphoreType.DMA,
            pltpu.SemaphoreType.DMA,
            pltpu.SemaphoreType.DMA,
        ],
    )


_agg_h1 = _make_agg(H1)
_agg_h2 = _make_agg(H2)


_R = 2048


def _tc_mm1_body(x_ref, w1_ref, h1_ref):
    h1_ref[...] = jnp.dot(x_ref[...], w1_ref[...],
                          preferred_element_type=jnp.float32)


def _tc_mm1(x, w1):
    return pl.pallas_call(
        _tc_mm1_body,
        grid=(NPAD // _R,),
        in_specs=[
            pl.BlockSpec((_R, D), lambda i: (i, 0)),
            pl.BlockSpec((D, H1), lambda i: (0, 0)),
        ],
        out_specs=pl.BlockSpec((_R, H1), lambda i: (i, 0)),
        out_shape=jax.ShapeDtypeStruct((NPAD, H1), jnp.float32),
    )(x, w1)


def _tc_scale_body(h1_ref, deg_ref, g1_ref, dinv_ref):
    deg = deg_ref[0, :] + deg_ref[1, :] + 1.0
    dinv = lax.rsqrt(deg)[:, None]
    g1_ref[...] = h1_ref[...] * dinv
    dinv_ref[...] = dinv


def _tc_scale(h1, deg):
    return pl.pallas_call(
        _tc_scale_body,
        grid=(NPAD // _R,),
        in_specs=[
            pl.BlockSpec((_R, H1), lambda i: (i, 0)),
            pl.BlockSpec((NC, _R), lambda i: (0, i)),
        ],
        out_specs=[
            pl.BlockSpec((_R, H1), lambda i: (i, 0)),
            pl.BlockSpec((_R, 1), lambda i: (i, 0)),
        ],
        out_shape=[
            jax.ShapeDtypeStruct((NPAD, H1), jnp.float32),
            jax.ShapeDtypeStruct((NPAD, 1), jnp.float32),
        ],
    )(h1, deg)


def _tc_d_body(agg_ref, g1_ref, dinv_ref, b1_ref, w2_ref, g2_ref):
    dinv = dinv_ref[...]
    tot = agg_ref[0] + agg_ref[1] + g1_ref[...]
    o1 = jnp.maximum(tot * dinv + b1_ref[...], 0.0)
    g2_ref[...] = jnp.dot(o1 * dinv, w2_ref[...],
                          preferred_element_type=jnp.float32)


def _tc_d(agg1, g1, dinv, b1, w2):
    return pl.pallas_call(
        _tc_d_body,
        grid=(NPAD // _R,),
        in_specs=[
            pl.BlockSpec((NC, _R, H1), lambda i: (0, i, 0)),
            pl.BlockSpec((_R, H1), lambda i: (i, 0)),
            pl.BlockSpec((_R, 1), lambda i: (i, 0)),
            pl.BlockSpec((1, H1), lambda i: (0, 0)),
            pl.BlockSpec((H1, H2), lambda i: (0, 0)),
        ],
        out_specs=pl.BlockSpec((_R, H2), lambda i: (i, 0)),
        out_shape=jax.ShapeDtypeStruct((NPAD, H2), jnp.float32),
    )(agg1, g1, dinv, b1, w2)


def _tc_f_body(agg_ref, g2_ref, dinv_ref, b2_ref, wfc_ref, bfc_ref, out_ref):
    dinv = dinv_ref[...]
    tot = agg_ref[0] + agg_ref[1] + g2_ref[...]
    o2 = jnp.maximum(tot * dinv + b2_ref[...], 0.0)
    y = jnp.dot(o2, wfc_ref[...], preferred_element_type=jnp.float32)
    out_ref[...] = jax.nn.sigmoid(y + bfc_ref[0, 0])


def _tc_f(agg2, g2, dinv, b2, wfc, bfc):
    return pl.pallas_call(
        _tc_f_body,
        grid=(NPAD // _R,),
        in_specs=[
            pl.BlockSpec((NC, _R, H2), lambda i: (0, i, 0)),
            pl.BlockSpec((_R, H2), lambda i: (i, 0)),
            pl.BlockSpec((_R, 1), lambda i: (i, 0)),
            pl.BlockSpec((1, H2), lambda i: (0, 0)),
            pl.BlockSpec((H2, 1), lambda i: (0, 0)),
            pl.BlockSpec((1, 1), lambda i: (0, 0), memory_space=pltpu.SMEM),
        ],
        out_specs=pl.BlockSpec((_R, 1), lambda i: (i, 0)),
        out_shape=jax.ShapeDtypeStruct((NPAD, 1), jnp.float32),
    )(agg2, g2, dinv, b2, wfc, bfc)


@jax.jit
def kernel(x, edge_index, W1, b1, W2, b2, Wfc, bfc):
    x_pad = jnp.zeros((NPAD, D), jnp.float32).at[:N].set(x)
    src = edge_index[0]
    dst = edge_index[1]
    h1 = _tc_mm1(x_pad, W1)
    deg = _deg_call(dst).reshape(NC, NPAD)
    g1, dinv = _tc_scale(h1, deg)
    agg1 = _agg_h1(g1, src, dst)
    g2 = _tc_d(agg1, g1, dinv, b1.reshape(1, H1), W2)
    agg2 = _agg_h2(g2, src, dst)
    out = _tc_f(agg2, g2, dinv, b2.reshape(1, H2), Wfc, bfc.reshape(1, 1))
    return out[:N]


# speedup vs baseline: 53.9752x; 1.0675x over previous
import functools

import jax
import jax.numpy as jnp
from jax import lax
from jax.experimental import pallas as pl
from jax.experimental.pallas import tpu as pltpu
from jax.experimental.pallas import tpu_sc as plsc

N = 10000
E = 320000
D = 128
H1 = 16
H2 = 64

NC = 2
NS = 16
NW = NC * NS

NPAD = 10240
RPT = NPAD // NS
EB = 128
CN = 78
NBUF = 6
NG = CN // NBUF
WBC = RPT // EB
IDXB = 16
NX = 4


def _zero_rows(ref, nrows, ncols):
    per_row = ncols // 16
    z = jnp.zeros((16,), jnp.float32)

    def body(t, carry):
        ref[t // per_row, pl.ds((t % per_row) * 16, 16)] = z
        return carry

    lax.fori_loop(0, nrows * per_row, body, 0)


def _edge_base(wid):
    return wid * (CN * EB) + jnp.minimum(wid, NX) * EB


def _preload_idx(ei_hbm, idx3, ebase, wid, sem):
    for k0 in range(0, CN, IDXB):
        descs = [
            pltpu.async_copy(ei_hbm.at[:, pl.ds(ebase + i * EB, EB)],
                             idx3.at[i], sem)
            for i in range(k0, min(k0 + IDXB, CN))
        ]
        for d in descs:
            d.wait()

    @pl.when(wid < NX)
    def _():
        pltpu.async_copy(ei_hbm.at[:, pl.ds(ebase + CN * EB, EB)],
                         idx3.at[CN], sem).wait()


def _deg_body(ei_hbm, out_hbm, idx3, ones_v, stage_v, acc,
              sem_i, ss0, ss1, ss2, ss3, ss4, ss5):
    sems = (ss0, ss1, ss2, ss3, ss4, ss5)
    c = lax.axis_index("c")
    s = lax.axis_index("s")
    row0 = s * RPT
    wid = c * NS + s
    ebase = _edge_base(wid)

    _preload_idx(ei_hbm, idx3, ebase, wid, sem_i)

    z = jnp.zeros((16,), jnp.float32)
    o = jnp.ones((16,), jnp.float32)
    for t in range(EB // 16):
        stage_v[pl.ds(t * 16, 16)] = z
        ones_v[pl.ds(t * 16, 16)] = o
    zd = [
        pltpu.async_copy(stage_v, acc.at[pl.ds(row0 + j * EB, EB)], sem_i)
        for j in range(WBC)
    ]
    for d in zd:
        d.wait()
    plsc.subcore_barrier()

    def grp(t, carry):
        for b in range(NBUF):
            i = t * NBUF + b

            @pl.when(t > 0)
            def _():
                pltpu.make_async_copy(ones_v, acc.at[idx3.at[i, 1]],
                                      sems[b]).wait()

            pltpu.async_copy(ones_v, acc.at[idx3.at[i, 1]], sems[b],
                             add=True)
        return carry

    lax.fori_loop(0, NG, grp, 0)
    for b in range(NBUF):
        pltpu.make_async_copy(ones_v, acc.at[idx3.at[b, 1]], sems[b]).wait()

    @pl.when(wid < NX)
    def _():
        pltpu.async_copy(ones_v, acc.at[idx3.at[CN, 1]], sems[0], add=True)
        pltpu.make_async_copy(ones_v, acc.at[idx3.at[CN, 1]], sems[0]).wait()

    plsc.subcore_barrier()

    for j in range(WBC):
        pltpu.sync_copy(acc.at[pl.ds(row0 + j * EB, EB)], stage_v)
        pltpu.sync_copy(stage_v,
                        out_hbm.at[pl.ds(c * NPAD + row0 + j * EB, EB)])


_deg_call = pl.kernel(
    _deg_body,
    out_type=jax.ShapeDtypeStruct((NC * NPAD,), jnp.float32),
    mesh=plsc.VectorSubcoreMesh(core_axis_name="c", subcore_axis_name="s"),
    compiler_params=pltpu.CompilerParams(use_tc_tiling_on_sc=False),
    scratch_types=[
        pltpu.VMEM((CN + 1, 2, EB), jnp.int32),
        pltpu.VMEM((EB,), jnp.float32),
        pltpu.VMEM((EB,), jnp.float32),
        pltpu.VMEM_SHARED((NPAD,), jnp.float32),
        pltpu.SemaphoreType.DMA,
        pltpu.SemaphoreType.DMA,
        pltpu.SemaphoreType.DMA,
        pltpu.SemaphoreType.DMA,
        pltpu.SemaphoreType.DMA,
        pltpu.SemaphoreType.DMA,
        pltpu.SemaphoreType.DMA,
    ],
)


def _make_agg(F):
    def body(g_hbm, ei_hbm, out_hbm,
             idx3, r0, r1, r2, r3, r4, r5, acc,
             sem_i, sg0, sg1, sg2, sg3, sg4, sg5,
             ss0, ss1, ss2, ss3, ss4, ss5):
        rows = (r0, r1, r2, r3, r4, r5)
        semg = (sg0, sg1, sg2, sg3, sg4, sg5)
        sems = (ss0, ss1, ss2, ss3, ss4, ss5)
        c = lax.axis_index("c")
        s = lax.axis_index("s")
        row0 = s * RPT
        wid = c * NS + s
        ebase = _edge_base(wid)

        _preload_idx(ei_hbm, idx3, ebase, wid, sem_i)

        _zero_rows(rows[0], EB, F)
        zd = [
            pltpu.async_copy(rows[0], acc.at[pl.ds(row0 + j * EB, EB)], sem_i)
            for j in range(WBC)
        ]
        for d in zd:
            d.wait()
        plsc.subcore_barrier()

        def grp(t, carry):
            for b in range(NBUF):
                i = t * NBUF + b

                @pl.when(t > 0)
                def _():
                    pltpu.make_async_copy(rows[b], acc.at[idx3.at[i, 1]],
                                          sems[b]).wait()

                pltpu.async_copy(g_hbm.at[idx3.at[i, 0]], rows[b], semg[b])
            for b in range(NBUF):
                i = t * NBUF + b
                pltpu.make_async_copy(g_hbm.at[idx3.at[i, 0]], rows[b],
                                      semg[b]).wait()
                pltpu.async_copy(rows[b], acc.at[idx3.at[i, 1]], sems[b],
                                 add=True)
            return carry

        lax.fori_loop(0, NG, grp, 0)
        for b in range(NBUF):
            pltpu.make_async_copy(rows[b], acc.at[idx3.at[b, 1]],
                                  sems[b]).wait()

        @pl.when(wid < NX)
        def _():
            pltpu.async_copy(g_hbm.at[idx3.at[CN, 0]], rows[0], semg[0])
            pltpu.make_async_copy(g_hbm.at[idx3.at[CN, 0]], rows[0],
                                  semg[0]).wait()
            pltpu.async_copy(rows[0], acc.at[idx3.at[CN, 1]], sems[0],
                             add=True)
            pltpu.make_async_copy(rows[0], acc.at[idx3.at[CN, 1]],
                                  sems[0]).wait()

        plsc.subcore_barrier()

        for j in range(WBC):
            pltpu.async_copy(acc.at[pl.ds(row0 + j * EB, EB)], rows[j],
                             semg[j])
        for j in range(WBC):
            pltpu.make_async_copy(acc.at[pl.ds(row0 + j * EB, EB)], rows[j],
                                  semg[j]).wait()
            pltpu.async_copy(rows[j],
                             out_hbm.at[c, pl.ds(row0 + j * EB, EB)], sems[j])
        for j in range(WBC):
            pltpu.make_async_copy(
                rows[j], out_hbm.at[c, pl.ds(row0, EB)], sems[j]).wait()

    return pl.kernel(
        body,
        out_type=jax.ShapeDtypeStruct((NC, NPAD, F), jnp.float32),
        mesh=plsc.VectorSubcoreMesh(core_axis_name="c", subcore_axis_name="s"),
        compiler_params=pltpu.CompilerParams(use_tc_tiling_on_sc=False),
        scratch_types=[
            pltpu.VMEM((CN + 1, 2, EB), jnp.int32),
            pltpu.VMEM((EB, F), jnp.float32),
            pltpu.VMEM((EB, F), jnp.float32),
            pltpu.VMEM((EB, F), jnp.float32),
            pltpu.VMEM((EB, F), jnp.float32),
            pltpu.VMEM((EB, F), jnp.float32),
            pltpu.VMEM((EB, F), jnp.float32),
            pltpu.VMEM_SHARED((NPAD, F), jnp.float32),
            pltpu.SemaphoreType.DMA,
            pltpu.SemaphoreType.DMA,
            pltpu.SemaphoreType.DMA,
            pltpu.SemaphoreType.DMA,
            pltpu.SemaphoreType.DMA,
            pltpu.SemaphoreType.DMA,
            pltpu.SemaphoreType.DMA,
            pltpu.SemaphoreType.DMA,
            pltpu.SemaphoreType.DMA,
            pltpu.SemaphoreType.DMA,
            pltpu.SemaphoreType.DMA,
            pltpu.SemaphoreType.DMA,
            pltpu.SemaphoreType.DMA,
        ],
    )


_agg_h1 = _make_agg(H1)
_agg_h2 = _make_agg(H2)


_R = 2048


def _tc_mm1_body(x_ref, w1_ref, h1_ref):
    h1_ref[...] = jnp.dot(x_ref[...], w1_ref[...],
                          preferred_element_type=jnp.float32)


def _tc_mm1(x, w1):
    return pl.pallas_call(
        _tc_mm1_body,
        grid=(NPAD // _R,),
        in_specs=[
            pl.BlockSpec((_R, D), lambda i: (i, 0)),
            pl.BlockSpec((D, H1), lambda i: (0, 0)),
        ],
        out_specs=pl.BlockSpec((_R, H1), lambda i: (i, 0)),
        out_shape=jax.ShapeDtypeStruct((NPAD, H1), jnp.float32),
    )(x, w1)


def _tc_scale_body(h1_ref, dega_ref, degb_ref, g1_ref, dinv_ref):
    deg = dega_ref[...] + degb_ref[...] + 1.0
    dinv = lax.rsqrt(deg)[:, None]
    g1_ref[...] = h1_ref[...] * dinv
    dinv_ref[...] = dinv


def _tc_scale(h1, deg_flat):
    return pl.pallas_call(
        _tc_scale_body,
        grid=(NPAD // _R,),
        in_specs=[
            pl.BlockSpec((_R, H1), lambda i: (i, 0)),
            pl.BlockSpec((_R,), lambda i: (i,)),
            pl.BlockSpec((_R,), lambda i: (i + NPAD // _R,)),
        ],
        out_specs=[
            pl.BlockSpec((_R, H1), lambda i: (i, 0)),
            pl.BlockSpec((_R, 1), lambda i: (i, 0)),
        ],
        out_shape=[
            jax.ShapeDtypeStruct((NPAD, H1), jnp.float32),
            jax.ShapeDtypeStruct((NPAD, 1), jnp.float32),
        ],
    )(h1, deg_flat, deg_flat)


def _tc_d_body(agg_ref, g1_ref, dinv_ref, b1_ref, w2_ref, g2_ref):
    dinv = dinv_ref[...]
    tot = agg_ref[0] + agg_ref[1] + g1_ref[...]
    o1 = jnp.maximum(tot * dinv + b1_ref[...], 0.0)
    g2_ref[...] = jnp.dot(o1 * dinv, w2_ref[...],
                          preferred_element_type=jnp.float32)


def _tc_d(agg1, g1, dinv, b1, w2):
    return pl.pallas_call(
        _tc_d_body,
        grid=(NPAD // _R,),
        in_specs=[
            pl.BlockSpec((NC, _R, H1), lambda i: (0, i, 0)),
            pl.BlockSpec((_R, H1), lambda i: (i, 0)),
            pl.BlockSpec((_R, 1), lambda i: (i, 0)),
            pl.BlockSpec((1, H1), lambda i: (0, 0)),
            pl.BlockSpec((H1, H2), lambda i: (0, 0)),
        ],
        out_specs=pl.BlockSpec((_R, H2), lambda i: (i, 0)),
        out_shape=jax.ShapeDtypeStruct((NPAD, H2), jnp.float32),
    )(agg1, g1, dinv, b1, w2)


def _tc_f_body(agg_ref, g2_ref, dinv_ref, b2_ref, wfc_ref, bfc_ref, out_ref):
    dinv = dinv_ref[...]
    tot = agg_ref[0] + agg_ref[1] + g2_ref[...]
    o2 = jnp.maximum(tot * dinv + b2_ref[...], 0.0)
    y = jnp.dot(o2, wfc_ref[...], preferred_element_type=jnp.float32)
    out_ref[...] = jax.nn.sigmoid(y + bfc_ref[0, 0])


def _tc_f(agg2, g2, dinv, b2, wfc, bfc):
    return pl.pallas_call(
        _tc_f_body,
        grid=(NPAD // _R,),
        in_specs=[
            pl.BlockSpec((NC, _R, H2), lambda i: (0, i, 0)),
            pl.BlockSpec((_R, H2), lambda i: (i, 0)),
            pl.BlockSpec((_R, 1), lambda i: (i, 0)),
            pl.BlockSpec((1, H2), lambda i: (0, 0)),
            pl.BlockSpec((H2, 1), lambda i: (0, 0)),
            pl.BlockSpec((1, 1), lambda i: (0, 0), memory_space=pltpu.SMEM),
        ],
        out_specs=pl.BlockSpec((_R, 1), lambda i: (i, 0)),
        out_shape=jax.ShapeDtypeStruct((NPAD, 1), jnp.float32),
    )(agg2, g2, dinv, b2, wfc, bfc)


@jax.jit
def kernel(x, edge_index, W1, b1, W2, b2, Wfc, bfc):
    x_pad = jnp.zeros((NPAD, D), jnp.float32).at[:N].set(x)
    h1 = _tc_mm1(x_pad, W1)
    deg = _deg_call(edge_index)
    g1, dinv = _tc_scale(h1, deg)
    agg1 = _agg_h1(g1, edge_index)
    g2 = _tc_d(agg1, g1, dinv, b1.reshape(1, H1), W2)
    agg2 = _agg_h2(g2, edge_index)
    out = _tc_f(agg2, g2, dinv, b2.reshape(1, H2), Wfc, bfc.reshape(1, 1))
    return out[:N]


# speedup vs baseline: 63.9280x; 1.1844x over previous
import functools

import jax
import jax.numpy as jnp
from jax import lax
from jax.experimental import pallas as pl
from jax.experimental.pallas import tpu as pltpu
from jax.experimental.pallas import tpu_sc as plsc

N = 10000
E = 320000
D = 128
H1 = 16
H2 = 64

NC = 2
NS = 16
NW = NC * NS

NPAD = 10240
RPT = NPAD // NS
EB = 128
CN = 78
NBUF = 6
NG = CN // NBUF
WBC = RPT // EB
IDXB = 16
NX = 4


def _zero_rows(ref, nrows, ncols):
    per_row = ncols // 16
    z = jnp.zeros((16,), jnp.float32)

    def body(t, carry):
        ref[t // per_row, pl.ds((t % per_row) * 16, 16)] = z
        return carry

    lax.fori_loop(0, nrows * per_row, body, 0)


def _edge_base(wid):
    return wid * (CN * EB) + jnp.minimum(wid, NX) * EB


def _preload_idx(ei_hbm, idx3, ebase, wid, sem):
    for k0 in range(0, CN, IDXB):
        descs = [
            pltpu.async_copy(ei_hbm.at[:, pl.ds(ebase + i * EB, EB)],
                             idx3.at[i], sem)
            for i in range(k0, min(k0 + IDXB, CN))
        ]
        for d in descs:
            d.wait()

    @pl.when(wid < NX)
    def _():
        pltpu.async_copy(ei_hbm.at[:, pl.ds(ebase + CN * EB, EB)],
                         idx3.at[CN], sem).wait()


def _deg_body(ei_hbm, out_hbm, idx3, ones_v, stage_v, acc,
              sem_i, ss0, ss1, ss2, ss3, ss4, ss5):
    sems = (ss0, ss1, ss2, ss3, ss4, ss5)
    c = lax.axis_index("c")
    s = lax.axis_index("s")
    row0 = s * RPT
    wid = c * NS + s
    ebase = _edge_base(wid)

    _preload_idx(ei_hbm, idx3, ebase, wid, sem_i)

    z = jnp.zeros((16,), jnp.float32)
    o = jnp.ones((16,), jnp.float32)
    for t in range(EB // 16):
        stage_v[pl.ds(t * 16, 16)] = z
        ones_v[pl.ds(t * 16, 16)] = o
    zd = [
        pltpu.async_copy(stage_v, acc.at[pl.ds(row0 + j * EB, EB)], sem_i)
        for j in range(WBC)
    ]
    for d in zd:
        d.wait()
    plsc.subcore_barrier()

    def grp(t, carry):
        for b in range(NBUF):
            i = t * NBUF + b

            @pl.when(t > 0)
            def _():
                pltpu.make_async_copy(ones_v, acc.at[idx3.at[i, 1]],
                                      sems[b]).wait()

            pltpu.async_copy(ones_v, acc.at[idx3.at[i, 1]], sems[b],
                             add=True)
        return carry

    lax.fori_loop(0, NG, grp, 0)
    for b in range(NBUF):
        pltpu.make_async_copy(ones_v, acc.at[idx3.at[b, 1]], sems[b]).wait()

    @pl.when(wid < NX)
    def _():
        pltpu.async_copy(ones_v, acc.at[idx3.at[CN, 1]], sems[0], add=True)
        pltpu.make_async_copy(ones_v, acc.at[idx3.at[CN, 1]], sems[0]).wait()

    plsc.subcore_barrier()

    for j in range(WBC):
        pltpu.sync_copy(acc.at[pl.ds(row0 + j * EB, EB)], stage_v)
        pltpu.sync_copy(stage_v,
                        out_hbm.at[pl.ds(c * NPAD + row0 + j * EB, EB)])


_deg_call = pl.kernel(
    _deg_body,
    out_type=jax.ShapeDtypeStruct((NC * NPAD,), jnp.float32),
    mesh=plsc.VectorSubcoreMesh(core_axis_name="c", subcore_axis_name="s"),
    compiler_params=pltpu.CompilerParams(use_tc_tiling_on_sc=False),
    scratch_types=[
        pltpu.VMEM((CN + 1, 2, EB), jnp.int32),
        pltpu.VMEM((EB,), jnp.float32),
        pltpu.VMEM((EB,), jnp.float32),
        pltpu.VMEM_SHARED((NPAD,), jnp.float32),
        pltpu.SemaphoreType.DMA,
        pltpu.SemaphoreType.DMA,
        pltpu.SemaphoreType.DMA,
        pltpu.SemaphoreType.DMA,
        pltpu.SemaphoreType.DMA,
        pltpu.SemaphoreType.DMA,
        pltpu.SemaphoreType.DMA,
    ],
)


def _make_agg(F):
    def body(g_hbm, ei_hbm, out_hbm,
             idx3, r0, r1, r2, r3, r4, r5, acc,
             sem_i, sg0, sg1, sg2, sg3, sg4, sg5,
             ss0, ss1, ss2, ss3, ss4, ss5):
        rows = (r0, r1, r2, r3, r4, r5)
        semg = (sg0, sg1, sg2, sg3, sg4, sg5)
        sems = (ss0, ss1, ss2, ss3, ss4, ss5)
        c = lax.axis_index("c")
        s = lax.axis_index("s")
        row0 = s * RPT
        wid = c * NS + s
        ebase = _edge_base(wid)

        _preload_idx(ei_hbm, idx3, ebase, wid, sem_i)

        _zero_rows(rows[0], EB, F)
        zd = [
            pltpu.async_copy(rows[0], acc.at[pl.ds(row0 + j * EB, EB)], sem_i)
            for j in range(WBC)
        ]
        for d in zd:
            d.wait()
        plsc.subcore_barrier()

        def grp(t, carry):
            for b in range(NBUF):
                i = t * NBUF + b

                @pl.when(t > 0)
                def _():
                    pltpu.make_async_copy(rows[b], acc.at[idx3.at[i, 1]],
                                          sems[b]).wait()

                pltpu.async_copy(g_hbm.at[idx3.at[i, 0]], rows[b], semg[b])
            for b in range(NBUF):
                i = t * NBUF + b
                pltpu.make_async_copy(g_hbm.at[idx3.at[i, 0]], rows[b],
                                      semg[b]).wait()
                pltpu.async_copy(rows[b], acc.at[idx3.at[i, 1]], sems[b],
                                 add=True)
            return carry

        lax.fori_loop(0, NG, grp, 0)
        for b in range(NBUF):
            pltpu.make_async_copy(rows[b], acc.at[idx3.at[b, 1]],
                                  sems[b]).wait()

        @pl.when(wid < NX)
        def _():
            pltpu.async_copy(g_hbm.at[idx3.at[CN, 0]], rows[0], semg[0])
            pltpu.make_async_copy(g_hbm.at[idx3.at[CN, 0]], rows[0],
                                  semg[0]).wait()
            pltpu.async_copy(rows[0], acc.at[idx3.at[CN, 1]], sems[0],
                             add=True)
            pltpu.make_async_copy(rows[0], acc.at[idx3.at[CN, 1]],
                                  sems[0]).wait()

        plsc.subcore_barrier()

        for j in range(WBC):
            pltpu.async_copy(acc.at[pl.ds(row0 + j * EB, EB)], rows[j],
                             semg[j])
        for j in range(WBC):
            pltpu.make_async_copy(acc.at[pl.ds(row0 + j * EB, EB)], rows[j],
                                  semg[j]).wait()
            pltpu.async_copy(rows[j],
                             out_hbm.at[c, pl.ds(row0 + j * EB, EB)], sems[j])
        for j in range(WBC):
            pltpu.make_async_copy(
                rows[j], out_hbm.at[c, pl.ds(row0, EB)], sems[j]).wait()

    return pl.kernel(
        body,
        out_type=jax.ShapeDtypeStruct((NC, NPAD, F), jnp.float32),
        mesh=plsc.VectorSubcoreMesh(core_axis_name="c", subcore_axis_name="s"),
        compiler_params=pltpu.CompilerParams(use_tc_tiling_on_sc=False),
        scratch_types=[
            pltpu.VMEM((CN + 1, 2, EB), jnp.int32),
            pltpu.VMEM((EB, F), jnp.float32),
            pltpu.VMEM((EB, F), jnp.float32),
            pltpu.VMEM((EB, F), jnp.float32),
            pltpu.VMEM((EB, F), jnp.float32),
            pltpu.VMEM((EB, F), jnp.float32),
            pltpu.VMEM((EB, F), jnp.float32),
            pltpu.VMEM_SHARED((NPAD, F), jnp.float32),
            pltpu.SemaphoreType.DMA,
            pltpu.SemaphoreType.DMA,
            pltpu.SemaphoreType.DMA,
            pltpu.SemaphoreType.DMA,
            pltpu.SemaphoreType.DMA,
            pltpu.SemaphoreType.DMA,
            pltpu.SemaphoreType.DMA,
            pltpu.SemaphoreType.DMA,
            pltpu.SemaphoreType.DMA,
            pltpu.SemaphoreType.DMA,
            pltpu.SemaphoreType.DMA,
            pltpu.SemaphoreType.DMA,
            pltpu.SemaphoreType.DMA,
        ],
    )


_agg_h1 = _make_agg(H1)
_agg_h2 = _make_agg(H2)


_R = 2048
P1 = 128 // H1
P2 = 128 // H2


def _tc_mm1_body(x_ref, w1p_ref, h1p_ref):
    h1p_ref[...] = jnp.dot(x_ref[...], w1p_ref[...],
                           preferred_element_type=jnp.float32)


def _tc_mm1(x_resh, w1p):
    return pl.pallas_call(
        _tc_mm1_body,
        grid=(NPAD // _R,),
        in_specs=[
            pl.BlockSpec((_R // P1, P1 * D), lambda i: (i, 0)),
            pl.BlockSpec((P1 * D, 128), lambda i: (0, 0)),
        ],
        out_specs=pl.BlockSpec((_R // P1, 128), lambda i: (i, 0)),
        out_shape=jax.ShapeDtypeStruct((NPAD // P1, 128), jnp.float32),
    )(x_resh, w1p)


def _tc_scale_body(h1p_ref, dega_ref, degb_ref, e16_ref, e64_ref,
                   g1p_ref, dinv16_ref, dinv64_ref):
    deg = dega_ref[0] + degb_ref[0] + 1.0
    dinv8 = lax.rsqrt(deg)
    dinv16 = jnp.dot(dinv8, e16_ref[...],
                     preferred_element_type=jnp.float32)
    g1p_ref[...] = h1p_ref[...] * dinv16
    dinv16_ref[...] = dinv16
    dinv64_ref[...] = jnp.dot(dinv8, e64_ref[...],
                              preferred_element_type=jnp.float32)


def _tc_scale(h1p, deg8, e16, e64):
    return pl.pallas_call(
        _tc_scale_body,
        grid=(NPAD // _R,),
        in_specs=[
            pl.BlockSpec((_R // P1, 128), lambda i: (i, 0)),
            pl.BlockSpec((1, _R // 8, 8), lambda i: (0, i, 0)),
            pl.BlockSpec((1, _R // 8, 8), lambda i: (1, i, 0)),
            pl.BlockSpec((8, 128), lambda i: (0, 0)),
            pl.BlockSpec((8, 512), lambda i: (0, 0)),
        ],
        out_specs=[
            pl.BlockSpec((_R // P1, 128), lambda i: (i, 0)),
            pl.BlockSpec((_R // P1, 128), lambda i: (i, 0)),
            pl.BlockSpec((_R // P1, 512), lambda i: (i, 0)),
        ],
        out_shape=[
            jax.ShapeDtypeStruct((NPAD // P1, 128), jnp.float32),
            jax.ShapeDtypeStruct((NPAD // P1, 128), jnp.float32),
            jax.ShapeDtypeStruct((NPAD // P1, 512), jnp.float32),
        ],
    )(h1p, deg8, deg8, e16, e64)


def _tc_d_body(agg_ref, g1p_ref, dinv16_ref, b1p_ref, w2p_ref, g2q_ref):
    dinv16 = dinv16_ref[...]
    tot = agg_ref[0] + agg_ref[1] + g1p_ref[...]
    o1 = jnp.maximum(tot * dinv16 + b1p_ref[...], 0.0)
    g2q_ref[...] = jnp.dot(o1 * dinv16, w2p_ref[...],
                           preferred_element_type=jnp.float32)


def _tc_d(agg1p, g1p, dinv16p, b1p, w2p):
    return pl.pallas_call(
        _tc_d_body,
        grid=(NPAD // _R,),
        in_specs=[
            pl.BlockSpec((NC, _R // P1, 128), lambda i: (0, i, 0)),
            pl.BlockSpec((_R // P1, 128), lambda i: (i, 0)),
            pl.BlockSpec((_R // P1, 128), lambda i: (i, 0)),
            pl.BlockSpec((1, 128), lambda i: (0, 0)),
            pl.BlockSpec((128, P1 * H2), lambda i: (0, 0)),
        ],
        out_specs=pl.BlockSpec((_R // P1, P1 * H2), lambda i: (i, 0)),
        out_shape=jax.ShapeDtypeStruct((NPAD // P1, P1 * H2), jnp.float32),
    )(agg1p, g1p, dinv16p, b1p, w2p)


def _tc_f_body(agg_ref, g2p_ref, dinv64_ref, b2p_ref, wfcp_ref, bfc_ref,
               out_ref):
    dinv64 = dinv64_ref[...]
    tot = agg_ref[0] + agg_ref[1] + g2p_ref[...]
    o2 = jnp.maximum(tot * dinv64 + b2p_ref[...], 0.0)
    y = jnp.dot(o2, wfcp_ref[...], preferred_element_type=jnp.float32)
    out_ref[...] = jax.nn.sigmoid(y + bfc_ref[0, 0])


def _tc_f(agg2p, g2p, dinv64p, b2p, wfcp, bfc):
    return pl.pallas_call(
        _tc_f_body,
        grid=(NPAD // _R,),
        in_specs=[
            pl.BlockSpec((NC, _R // P2, 128), lambda i: (0, i, 0)),
            pl.BlockSpec((_R // P2, 128), lambda i: (i, 0)),
            pl.BlockSpec((_R // P2, 128), lambda i: (i, 0)),
            pl.BlockSpec((1, 128), lambda i: (0, 0)),
            pl.BlockSpec((128, P2), lambda i: (0, 0)),
            pl.BlockSpec((1, 1), lambda i: (0, 0), memory_space=pltpu.SMEM),
        ],
        out_specs=pl.BlockSpec((_R // P2, P2), lambda i: (i, 0)),
        out_shape=jax.ShapeDtypeStruct((NPAD // P2, P2), jnp.float32),
    )(agg2p, g2p, dinv64p, b2p, wfcp, bfc)


def _block_diag(w, k):
    a, b = w.shape
    eye = jnp.eye(k, dtype=w.dtype)
    return (eye[:, None, :, None] * w[None, :, None, :]).reshape(k * a, k * b)


@jax.jit
def kernel(x, edge_index, W1, b1, W2, b2, Wfc, bfc):
    x_pad = jnp.zeros((NPAD, D), jnp.float32).at[:N].set(x)
    x_resh = x_pad.reshape(NPAD // P1, P1 * D)
    w1p = _block_diag(W1, P1)
    w2p = _block_diag(W2, P1)
    wfcp = _block_diag(Wfc, P2)
    b1p = jnp.tile(b1, P1).reshape(1, 128)
    b2p = jnp.tile(b2, P2).reshape(1, 128)

    e16 = (jnp.arange(8)[:, None] ==
           jnp.arange(128)[None, :] // 16).astype(jnp.float32)
    e64 = (jnp.arange(8)[:, None] ==
           jnp.arange(512)[None, :] // 64).astype(jnp.float32)

    deg = _deg_call(edge_index)
    h1p = _tc_mm1(x_resh, w1p)
    deg8 = deg.reshape(NC, NPAD // 8, 8)
    g1p, dinv16p, dinv64q = _tc_scale(h1p, deg8, e16, e64)
    agg1 = _agg_h1(g1p.reshape(NPAD, H1), edge_index)
    agg1p = agg1.reshape(NC, NPAD // P1, 128)
    g2q = _tc_d(agg1p, g1p, dinv16p, b1p, w2p)
    agg2 = _agg_h2(g2q.reshape(NPAD, H2), edge_index)
    agg2p = agg2.reshape(NC, NPAD // P2, 128)
    g2p = g2q.reshape(NPAD // P2, 128)
    dinv64p = dinv64q.reshape(NPAD // P2, 128)
    outp = _tc_f(agg2p, g2p, dinv64p, b2p, wfcp, bfc.reshape(1, 1))
    return outp.reshape(NPAD, 1)[:N]


# speedup vs baseline: 65.8833x; 1.0306x over previous
import functools

import jax
import jax.numpy as jnp
from jax import lax
from jax.experimental import pallas as pl
from jax.experimental.pallas import tpu as pltpu
from jax.experimental.pallas import tpu_sc as plsc

N = 10000
E = 320000
D = 128
H1 = 16
H2 = 64

NC = 2
NS = 16
NW = NC * NS

NPAD = 10240
RPT = NPAD // NS
EB = 128
CN = 78
NBUF = 6
NG = CN // NBUF
NBA = 8
NGA = CN // NBA
WBC = RPT // EB
IDXB = 32
NX = 4


def _zero_rows(ref, nrows, ncols):
    per_row = ncols // 16
    z = jnp.zeros((16,), jnp.float32)

    def body(t, carry):
        ref[t // per_row, pl.ds((t % per_row) * 16, 16)] = z
        return carry

    lax.fori_loop(0, nrows * per_row, body, 0)


def _edge_base(wid):
    return wid * (CN * EB) + jnp.minimum(wid, NX) * EB


def _preload_idx(ei_hbm, idx3, ebase, wid, sem):
    for k0 in range(0, CN, IDXB):
        descs = [
            pltpu.async_copy(ei_hbm.at[:, pl.ds(ebase + i * EB, EB)],
                             idx3.at[i], sem)
            for i in range(k0, min(k0 + IDXB, CN))
        ]
        for d in descs:
            d.wait()

    @pl.when(wid < NX)
    def _():
        pltpu.async_copy(ei_hbm.at[:, pl.ds(ebase + CN * EB, EB)],
                         idx3.at[CN], sem).wait()


def _deg_body(ei_hbm, out_hbm, idx3, ones_v, stage_v, acc,
              sem_i, ss0, ss1, ss2, ss3, ss4, ss5):
    sems = (ss0, ss1, ss2, ss3, ss4, ss5)
    c = lax.axis_index("c")
    s = lax.axis_index("s")
    row0 = s * RPT
    wid = c * NS + s
    ebase = _edge_base(wid)

    _preload_idx(ei_hbm, idx3, ebase, wid, sem_i)

    z = jnp.zeros((16,), jnp.float32)
    o = jnp.ones((16,), jnp.float32)
    for t in range(EB // 16):
        stage_v[pl.ds(t * 16, 16)] = z
        ones_v[pl.ds(t * 16, 16)] = o
    zd = [
        pltpu.async_copy(stage_v, acc.at[pl.ds(row0 + j * EB, EB)], sem_i)
        for j in range(WBC)
    ]
    for d in zd:
        d.wait()
    plsc.subcore_barrier()

    def grp(t, carry):
        for b in range(NBUF):
            i = t * NBUF + b

            @pl.when(t > 0)
            def _():
                pltpu.make_async_copy(ones_v, acc.at[idx3.at[i, 1]],
                                      sems[b]).wait()

            pltpu.async_copy(ones_v, acc.at[idx3.at[i, 1]], sems[b],
                             add=True)
        return carry

    lax.fori_loop(0, NG, grp, 0)
    for b in range(NBUF):
        pltpu.make_async_copy(ones_v, acc.at[idx3.at[b, 1]], sems[b]).wait()

    @pl.when(wid < NX)
    def _():
        pltpu.async_copy(ones_v, acc.at[idx3.at[CN, 1]], sems[0], add=True)
        pltpu.make_async_copy(ones_v, acc.at[idx3.at[CN, 1]], sems[0]).wait()

    plsc.subcore_barrier()

    for j in range(WBC):
        pltpu.sync_copy(acc.at[pl.ds(row0 + j * EB, EB)], stage_v)
        pltpu.sync_copy(stage_v,
                        out_hbm.at[pl.ds(c * NPAD + row0 + j * EB, EB)])


_deg_call = pl.kernel(
    _deg_body,
    out_type=jax.ShapeDtypeStruct((NC * NPAD,), jnp.float32),
    mesh=plsc.VectorSubcoreMesh(core_axis_name="c", subcore_axis_name="s"),
    compiler_params=pltpu.CompilerParams(use_tc_tiling_on_sc=True),
    scratch_types=[
        pltpu.VMEM((CN + 1, 2, EB), jnp.int32),
        pltpu.VMEM((EB,), jnp.float32),
        pltpu.VMEM((EB,), jnp.float32),
        pltpu.VMEM_SHARED((NPAD,), jnp.float32),
        pltpu.SemaphoreType.DMA,
        pltpu.SemaphoreType.DMA,
        pltpu.SemaphoreType.DMA,
        pltpu.SemaphoreType.DMA,
        pltpu.SemaphoreType.DMA,
        pltpu.SemaphoreType.DMA,
        pltpu.SemaphoreType.DMA,
    ],
)


def _make_agg(F):
    def body(g_hbm, ei_hbm, out_hbm,
             idx3, r0, r1, r2, r3, r4, r5, r6, r7, acc,
             sem_i, sg0, sg1, sg2, sg3, sg4, sg5, sg6, sg7,
             ss0, ss1, ss2, ss3, ss4, ss5, ss6, ss7):
        rows = (r0, r1, r2, r3, r4, r5, r6, r7)
        semg = (sg0, sg1, sg2, sg3, sg4, sg5, sg6, sg7)
        sems = (ss0, ss1, ss2, ss3, ss4, ss5, ss6, ss7)
        c = lax.axis_index("c")
        s = lax.axis_index("s")
        row0 = s * RPT
        wid = c * NS + s
        ebase = _edge_base(wid)

        _preload_idx(ei_hbm, idx3, ebase, wid, sem_i)

        _zero_rows(rows[0], EB, F)
        zd = [
            pltpu.async_copy(rows[0], acc.at[pl.ds(row0 + j * EB, EB)], sem_i)
            for j in range(WBC)
        ]
        for d in zd:
            d.wait()
        plsc.subcore_barrier()

        def grp(t, carry):
            for b in range(NBA):
                i = t * NBA + b

                @pl.when(t > 0)
                def _():
                    pltpu.make_async_copy(rows[b], acc.at[idx3.at[i, 1]],
                                          sems[b]).wait()

                pltpu.async_copy(g_hbm.at[idx3.at[i, 0]], rows[b], semg[b])
            for b in range(NBA):
                i = t * NBA + b
                pltpu.make_async_copy(g_hbm.at[idx3.at[i, 0]], rows[b],
                                      semg[b]).wait()
                pltpu.async_copy(rows[b], acc.at[idx3.at[i, 1]], sems[b],
                                 add=True)
            return carry

        lax.fori_loop(0, NGA, grp, 0)
        ntail = CN - NGA * NBA
        for b in range(ntail):
            i = NGA * NBA + b
            pltpu.make_async_copy(rows[b], acc.at[idx3.at[i, 1]],
                                  sems[b]).wait()
            pltpu.async_copy(g_hbm.at[idx3.at[i, 0]], rows[b], semg[b])
        for b in range(ntail):
            i = NGA * NBA + b
            pltpu.make_async_copy(g_hbm.at[idx3.at[i, 0]], rows[b],
                                  semg[b]).wait()
            pltpu.async_copy(rows[b], acc.at[idx3.at[i, 1]], sems[b],
                             add=True)
        for b in range(NBA):
            pltpu.make_async_copy(rows[b], acc.at[idx3.at[b, 1]],
                                  sems[b]).wait()

        @pl.when(wid < NX)
        def _():
            pltpu.async_copy(g_hbm.at[idx3.at[CN, 0]], rows[0], semg[0])
            pltpu.make_async_copy(g_hbm.at[idx3.at[CN, 0]], rows[0],
                                  semg[0]).wait()
            pltpu.async_copy(rows[0], acc.at[idx3.at[CN, 1]], sems[0],
                             add=True)
            pltpu.make_async_copy(rows[0], acc.at[idx3.at[CN, 1]],
                                  sems[0]).wait()

        plsc.subcore_barrier()

        for j in range(WBC):
            pltpu.async_copy(acc.at[pl.ds(row0 + j * EB, EB)], rows[j],
                             semg[j])
        for j in range(WBC):
            pltpu.make_async_copy(acc.at[pl.ds(row0 + j * EB, EB)], rows[j],
                                  semg[j]).wait()
            pltpu.async_copy(rows[j],
                             out_hbm.at[c, pl.ds(row0 + j * EB, EB)], sems[j])
        for j in range(WBC):
            pltpu.make_async_copy(
                rows[j], out_hbm.at[c, pl.ds(row0, EB)], sems[j]).wait()

    return pl.kernel(
        body,
        out_type=jax.ShapeDtypeStruct((NC, NPAD, F), jnp.float32),
        mesh=plsc.VectorSubcoreMesh(core_axis_name="c", subcore_axis_name="s"),
        compiler_params=pltpu.CompilerParams(use_tc_tiling_on_sc=False),
        scratch_types=[
            pltpu.VMEM((CN + 1, 2, EB), jnp.int32),
            pltpu.VMEM((EB, F), jnp.float32),
            pltpu.VMEM((EB, F), jnp.float32),
            pltpu.VMEM((EB, F), jnp.float32),
            pltpu.VMEM((EB, F), jnp.float32),
            pltpu.VMEM((EB, F), jnp.float32),
            pltpu.VMEM((EB, F), jnp.float32),
            pltpu.VMEM((EB, F), jnp.float32),
            pltpu.VMEM((EB, F), jnp.float32),
            pltpu.VMEM_SHARED((NPAD, F), jnp.float32),
        ] + [pltpu.SemaphoreType.DMA] * 17,
    )


_agg_h1 = _make_agg(H1)
_agg_h2 = _make_agg(H2)


_R = 2048
P1 = 128 // H1
P2 = 128 // H2


def _tc_mm1_body(x_ref, w1p_ref, h1p_ref):
    h1p_ref[...] = jnp.dot(x_ref[...], w1p_ref[...],
                           preferred_element_type=jnp.float32)


def _tc_mm1(x_resh, w1p):
    return pl.pallas_call(
        _tc_mm1_body,
        grid=(NPAD // _R,),
        in_specs=[
            pl.BlockSpec((_R // P1, P1 * D), lambda i: (i, 0)),
            pl.BlockSpec((P1 * D, 128), lambda i: (0, 0)),
        ],
        out_specs=pl.BlockSpec((_R // P1, 128), lambda i: (i, 0)),
        out_shape=jax.ShapeDtypeStruct((NPAD // P1, 128), jnp.float32),
    )(x_resh, w1p)


def _tc_scale_body(h1p_ref, dega_ref, degb_ref, e16_ref, e64_ref,
                   g1p_ref, dinv16_ref, dinv64_ref):
    deg = dega_ref[0] + degb_ref[0] + 1.0
    dinv8 = lax.rsqrt(deg)
    dinv16 = jnp.dot(dinv8, e16_ref[...],
                     preferred_element_type=jnp.float32)
    g1p_ref[...] = h1p_ref[...] * dinv16
    dinv16_ref[...] = dinv16
    dinv64_ref[...] = jnp.dot(dinv8, e64_ref[...],
                              preferred_element_type=jnp.float32)


def _tc_scale(h1p, deg8, e16, e64):
    return pl.pallas_call(
        _tc_scale_body,
        grid=(NPAD // _R,),
        in_specs=[
            pl.BlockSpec((_R // P1, 128), lambda i: (i, 0)),
            pl.BlockSpec((1, _R // 8, 8), lambda i: (0, i, 0)),
            pl.BlockSpec((1, _R // 8, 8), lambda i: (1, i, 0)),
            pl.BlockSpec((8, 128), lambda i: (0, 0)),
            pl.BlockSpec((8, 512), lambda i: (0, 0)),
        ],
        out_specs=[
            pl.BlockSpec((_R // P1, 128), lambda i: (i, 0)),
            pl.BlockSpec((_R // P1, 128), lambda i: (i, 0)),
            pl.BlockSpec((_R // P1, 512), lambda i: (i, 0)),
        ],
        out_shape=[
            jax.ShapeDtypeStruct((NPAD // P1, 128), jnp.float32),
            jax.ShapeDtypeStruct((NPAD // P1, 128), jnp.float32),
            jax.ShapeDtypeStruct((NPAD // P1, 512), jnp.float32),
        ],
    )(h1p, deg8, deg8, e16, e64)


def _tc_d_body(agg_ref, g1p_ref, dinv16_ref, b1p_ref, w2p_ref, g2q_ref):
    dinv16 = dinv16_ref[...]
    tot = agg_ref[0] + agg_ref[1] + g1p_ref[...]
    o1 = jnp.maximum(tot * dinv16 + b1p_ref[...], 0.0)
    g2q_ref[...] = jnp.dot(o1 * dinv16, w2p_ref[...],
                           preferred_element_type=jnp.float32)


def _tc_d(agg1p, g1p, dinv16p, b1p, w2p):
    return pl.pallas_call(
        _tc_d_body,
        grid=(NPAD // _R,),
        in_specs=[
            pl.BlockSpec((NC, _R // P1, 128), lambda i: (0, i, 0)),
            pl.BlockSpec((_R // P1, 128), lambda i: (i, 0)),
            pl.BlockSpec((_R // P1, 128), lambda i: (i, 0)),
            pl.BlockSpec((1, 128), lambda i: (0, 0)),
            pl.BlockSpec((128, P1 * H2), lambda i: (0, 0)),
        ],
        out_specs=pl.BlockSpec((_R // P1, P1 * H2), lambda i: (i, 0)),
        out_shape=jax.ShapeDtypeStruct((NPAD // P1, P1 * H2), jnp.float32),
    )(agg1p, g1p, dinv16p, b1p, w2p)


def _tc_f_body(agg_ref, g2p_ref, dinv64_ref, b2p_ref, wfcp_ref, bfc_ref,
               out_ref):
    dinv64 = dinv64_ref[...]
    tot = agg_ref[0] + agg_ref[1] + g2p_ref[...]
    o2 = jnp.maximum(tot * dinv64 + b2p_ref[...], 0.0)
    y = jnp.dot(o2, wfcp_ref[...], preferred_element_type=jnp.float32)
    out_ref[...] = jax.nn.sigmoid(y + bfc_ref[0, 0])


def _tc_f(agg2p, g2p, dinv64p, b2p, wfcp, bfc):
    return pl.pallas_call(
        _tc_f_body,
        grid=(NPAD // _R,),
        in_specs=[
            pl.BlockSpec((NC, _R // P2, 128), lambda i: (0, i, 0)),
            pl.BlockSpec((_R // P2, 128), lambda i: (i, 0)),
            pl.BlockSpec((_R // P2, 128), lambda i: (i, 0)),
            pl.BlockSpec((1, 128), lambda i: (0, 0)),
            pl.BlockSpec((128, P2), lambda i: (0, 0)),
            pl.BlockSpec((1, 1), lambda i: (0, 0), memory_space=pltpu.SMEM),
        ],
        out_specs=pl.BlockSpec((_R // P2, P2), lambda i: (i, 0)),
        out_shape=jax.ShapeDtypeStruct((NPAD // P2, P2), jnp.float32),
    )(agg2p, g2p, dinv64p, b2p, wfcp, bfc)


def _block_diag(w, k):
    a, b = w.shape
    eye = jnp.eye(k, dtype=w.dtype)
    return (eye[:, None, :, None] * w[None, :, None, :]).reshape(k * a, k * b)


@jax.jit
def kernel(x, edge_index, W1, b1, W2, b2, Wfc, bfc):
    x_pad = jnp.zeros((NPAD, D), jnp.float32).at[:N].set(x)
    x_resh = x_pad.reshape(NPAD // P1, P1 * D)
    w1p = _block_diag(W1, P1)
    w2p = _block_diag(W2, P1)
    wfcp = _block_diag(Wfc, P2)
    b1p = jnp.tile(b1, P1).reshape(1, 128)
    b2p = jnp.tile(b2, P2).reshape(1, 128)

    e16 = (jnp.arange(8)[:, None] ==
           jnp.arange(128)[None, :] // 16).astype(jnp.float32)
    e64 = (jnp.arange(8)[:, None] ==
           jnp.arange(512)[None, :] // 64).astype(jnp.float32)

    deg = _deg_call(edge_index)
    h1p = _tc_mm1(x_resh, w1p)
    deg8 = deg.reshape(NC, NPAD // 8, 8)
    g1p, dinv16p, dinv64q = _tc_scale(h1p, deg8, e16, e64)
    agg1 = _agg_h1(g1p.reshape(NPAD, H1), edge_index)
    agg1p = agg1.reshape(NC, NPAD // P1, 128)
    g2q = _tc_d(agg1p, g1p, dinv16p, b1p, w2p)
    agg2 = _agg_h2(g2q.reshape(NPAD, H2), edge_index)
    agg2p = agg2.reshape(NC, NPAD // P2, 128)
    g2p = g2q.reshape(NPAD // P2, 128)
    dinv64p = dinv64q.reshape(NPAD // P2, 128)
    outp = _tc_f(agg2p, g2p, dinv64p, b2p, wfcp, bfc.reshape(1, 1))
    return outp.reshape(NPAD, 1)[:N]


# speedup vs baseline: 92.2879x; 1.4008x over previous
import functools

import jax
import jax.numpy as jnp
from jax import lax
from jax.experimental import pallas as pl
from jax.experimental.pallas import tpu as pltpu
from jax.experimental.pallas import tpu_sc as plsc

N = 10000
E = 320000
D = 128
H1 = 16
H2 = 64

NC = 2
NS = 16
NW = NC * NS

NPAD = 10240
RPT = NPAD // NS
EB = 128
CN = 78
NBUF = 6
NG = CN // NBUF
NBA = 8
NGA = CN // NBA
WBC = RPT // EB
IDXB = 32
NX = 4


def _zero_rows(ref, nrows, ncols):
    per_row = ncols // 16
    z = jnp.zeros((16,), jnp.float32)

    def body(t, carry):
        ref[t // per_row, pl.ds((t % per_row) * 16, 16)] = z
        return carry

    lax.fori_loop(0, nrows * per_row, body, 0)


def _edge_base(wid):
    return wid * (CN * EB) + jnp.minimum(wid, NX) * EB


def _preload_idx(ei_hbm, idx3, ebase, wid, sem):
    for k0 in range(0, CN, IDXB):
        descs = [
            pltpu.async_copy(ei_hbm.at[:, pl.ds(ebase + i * EB, EB)],
                             idx3.at[i], sem)
            for i in range(k0, min(k0 + IDXB, CN))
        ]
        for d in descs:
            d.wait()

    @pl.when(wid < NX)
    def _():
        pltpu.async_copy(ei_hbm.at[:, pl.ds(ebase + CN * EB, EB)],
                         idx3.at[CN], sem).wait()


def _deg_body(ei_hbm, out_hbm, idx3, ones_v, stage_v, acc,
              sem_i, ss0, ss1, ss2, ss3, ss4, ss5):
    sems = (ss0, ss1, ss2, ss3, ss4, ss5)
    c = lax.axis_index("c")
    s = lax.axis_index("s")
    row0 = s * RPT
    wid = c * NS + s
    ebase = _edge_base(wid)

    _preload_idx(ei_hbm, idx3, ebase, wid, sem_i)

    z = jnp.zeros((16,), jnp.float32)
    o = jnp.ones((16,), jnp.float32)
    for t in range(EB // 16):
        stage_v[pl.ds(t * 16, 16)] = z
        ones_v[pl.ds(t * 16, 16)] = o
    zd = [
        pltpu.async_copy(stage_v, acc.at[pl.ds(row0 + j * EB, EB)], sem_i)
        for j in range(WBC)
    ]
    for d in zd:
        d.wait()
    plsc.subcore_barrier()

    def grp(t, carry):
        for b in range(NBUF):
            i = t * NBUF + b

            @pl.when(t > 0)
            def _():
                pltpu.make_async_copy(ones_v, acc.at[idx3.at[i, 1]],
                                      sems[b]).wait()

            pltpu.async_copy(ones_v, acc.at[idx3.at[i, 1]], sems[b],
                             add=True)
        return carry

    lax.fori_loop(0, NG, grp, 0)
    for b in range(NBUF):
        pltpu.make_async_copy(ones_v, acc.at[idx3.at[b, 1]], sems[b]).wait()

    @pl.when(wid < NX)
    def _():
        pltpu.async_copy(ones_v, acc.at[idx3.at[CN, 1]], sems[0], add=True)
        pltpu.make_async_copy(ones_v, acc.at[idx3.at[CN, 1]], sems[0]).wait()

    plsc.subcore_barrier()

    for j in range(WBC):
        pltpu.sync_copy(acc.at[pl.ds(row0 + j * EB, EB)], stage_v)
        pltpu.sync_copy(stage_v,
                        out_hbm.at[pl.ds(c * NPAD + row0 + j * EB, EB)])


_deg_call = pl.kernel(
    _deg_body,
    out_type=jax.ShapeDtypeStruct((NC * NPAD,), jnp.float32),
    mesh=plsc.VectorSubcoreMesh(core_axis_name="c", subcore_axis_name="s"),
    compiler_params=pltpu.CompilerParams(use_tc_tiling_on_sc=True),
    scratch_types=[
        pltpu.VMEM((CN + 1, 2, EB), jnp.int32),
        pltpu.VMEM((EB,), jnp.float32),
        pltpu.VMEM((EB,), jnp.float32),
        pltpu.VMEM_SHARED((NPAD,), jnp.float32),
        pltpu.SemaphoreType.DMA,
        pltpu.SemaphoreType.DMA,
        pltpu.SemaphoreType.DMA,
        pltpu.SemaphoreType.DMA,
        pltpu.SemaphoreType.DMA,
        pltpu.SemaphoreType.DMA,
        pltpu.SemaphoreType.DMA,
    ],
)


def _make_agg(F):
    def body(g_hbm, ei_hbm, out_hbm,
             idx3, r0, r1, r2, r3, r4, r5, r6, r7, acc,
             sem_i, sg0, sg1, sg2, sg3, sg4, sg5, sg6, sg7,
             ss0, ss1, ss2, ss3, ss4, ss5, ss6, ss7):
        rows = (r0, r1, r2, r3, r4, r5, r6, r7)
        semg = (sg0, sg1, sg2, sg3, sg4, sg5, sg6, sg7)
        sems = (ss0, ss1, ss2, ss3, ss4, ss5, ss6, ss7)
        c = lax.axis_index("c")
        s = lax.axis_index("s")
        row0 = s * RPT
        wid = c * NS + s
        ebase = _edge_base(wid)

        _preload_idx(ei_hbm, idx3, ebase, wid, sem_i)

        _zero_rows(rows[0], EB, F)
        zd = [
            pltpu.async_copy(rows[0], acc.at[pl.ds(row0 + j * EB, EB)], sem_i)
            for j in range(WBC)
        ]
        for d in zd:
            d.wait()
        plsc.subcore_barrier()

        def grp(t, carry):
            for b in range(NBA):
                i = t * NBA + b

                @pl.when(t > 0)
                def _():
                    pltpu.make_async_copy(rows[b], acc.at[idx3.at[i, 1]],
                                          sems[b]).wait()

                pltpu.async_copy(g_hbm.at[idx3.at[i, 0]], rows[b], semg[b])
            for b in range(NBA):
                i = t * NBA + b
                pltpu.make_async_copy(g_hbm.at[idx3.at[i, 0]], rows[b],
                                      semg[b]).wait()
                pltpu.async_copy(rows[b], acc.at[idx3.at[i, 1]], sems[b],
                                 add=True)
            return carry

        lax.fori_loop(0, NGA, grp, 0)
        ntail = CN - NGA * NBA
        for b in range(ntail):
            i = NGA * NBA + b
            pltpu.make_async_copy(rows[b], acc.at[idx3.at[i, 1]],
                                  sems[b]).wait()
            pltpu.async_copy(g_hbm.at[idx3.at[i, 0]], rows[b], semg[b])
        for b in range(ntail):
            i = NGA * NBA + b
            pltpu.make_async_copy(g_hbm.at[idx3.at[i, 0]], rows[b],
                                  semg[b]).wait()
            pltpu.async_copy(rows[b], acc.at[idx3.at[i, 1]], sems[b],
                             add=True)
        for b in range(NBA):
            pltpu.make_async_copy(rows[b], acc.at[idx3.at[b, 1]],
                                  sems[b]).wait()

        @pl.when(wid < NX)
        def _():
            pltpu.async_copy(g_hbm.at[idx3.at[CN, 0]], rows[0], semg[0])
            pltpu.make_async_copy(g_hbm.at[idx3.at[CN, 0]], rows[0],
                                  semg[0]).wait()
            pltpu.async_copy(rows[0], acc.at[idx3.at[CN, 1]], sems[0],
                             add=True)
            pltpu.make_async_copy(rows[0], acc.at[idx3.at[CN, 1]],
                                  sems[0]).wait()

        plsc.subcore_barrier()

        for j in range(WBC):
            pltpu.async_copy(acc.at[pl.ds(row0 + j * EB, EB)], rows[j],
                             semg[j])
        for j in range(WBC):
            pltpu.make_async_copy(acc.at[pl.ds(row0 + j * EB, EB)], rows[j],
                                  semg[j]).wait()
            pltpu.async_copy(rows[j],
                             out_hbm.at[c, pl.ds(row0 + j * EB, EB)], sems[j])
        for j in range(WBC):
            pltpu.make_async_copy(
                rows[j], out_hbm.at[c, pl.ds(row0, EB)], sems[j]).wait()

    return pl.kernel(
        body,
        out_type=jax.ShapeDtypeStruct((NC, NPAD, F), jnp.float32),
        mesh=plsc.VectorSubcoreMesh(core_axis_name="c", subcore_axis_name="s"),
        compiler_params=pltpu.CompilerParams(use_tc_tiling_on_sc=False),
        scratch_types=[
            pltpu.VMEM((CN + 1, 2, EB), jnp.int32),
            pltpu.VMEM((EB, F), jnp.float32),
            pltpu.VMEM((EB, F), jnp.float32),
            pltpu.VMEM((EB, F), jnp.float32),
            pltpu.VMEM((EB, F), jnp.float32),
            pltpu.VMEM((EB, F), jnp.float32),
            pltpu.VMEM((EB, F), jnp.float32),
            pltpu.VMEM((EB, F), jnp.float32),
            pltpu.VMEM((EB, F), jnp.float32),
            pltpu.VMEM_SHARED((NPAD, F), jnp.float32),
        ] + [pltpu.SemaphoreType.DMA] * 17,
    )


_agg_h1 = _make_agg(H1)


_R = 2048
P1 = 128 // H1
P2 = 128 // H2


def _tc_mm1_body(x_ref, w1p_ref, h1p_ref):
    h1p_ref[...] = jnp.dot(x_ref[...], w1p_ref[...],
                           preferred_element_type=jnp.float32)


def _tc_mm1(x_resh, w1p):
    return pl.pallas_call(
        _tc_mm1_body,
        grid=(NPAD // _R,),
        in_specs=[
            pl.BlockSpec((_R // P1, P1 * D), lambda i: (i, 0)),
            pl.BlockSpec((P1 * D, 128), lambda i: (0, 0)),
        ],
        out_specs=pl.BlockSpec((_R // P1, 128), lambda i: (i, 0)),
        out_shape=jax.ShapeDtypeStruct((NPAD // P1, 128), jnp.float32),
    )(x_resh, w1p)


def _tc_scale_body(h1p_ref, dega_ref, degb_ref, e16_ref, e64_ref,
                   g1p_ref, dinv16_ref, dinv64_ref):
    deg = dega_ref[0] + degb_ref[0] + 1.0
    dinv8 = lax.rsqrt(deg)
    dinv16 = jnp.dot(dinv8, e16_ref[...],
                     preferred_element_type=jnp.float32)
    g1p_ref[...] = h1p_ref[...] * dinv16
    dinv16_ref[...] = dinv16
    dinv64_ref[...] = jnp.dot(dinv8, e64_ref[...],
                              preferred_element_type=jnp.float32)


def _tc_scale(h1p, deg8, e16, e64):
    return pl.pallas_call(
        _tc_scale_body,
        grid=(NPAD // _R,),
        in_specs=[
            pl.BlockSpec((_R // P1, 128), lambda i: (i, 0)),
            pl.BlockSpec((1, _R // 8, 8), lambda i: (0, i, 0)),
            pl.BlockSpec((1, _R // 8, 8), lambda i: (1, i, 0)),
            pl.BlockSpec((8, 128), lambda i: (0, 0)),
            pl.BlockSpec((8, 512), lambda i: (0, 0)),
        ],
        out_specs=[
            pl.BlockSpec((_R // P1, 128), lambda i: (i, 0)),
            pl.BlockSpec((_R // P1, 128), lambda i: (i, 0)),
            pl.BlockSpec((_R // P1, 512), lambda i: (i, 0)),
        ],
        out_shape=[
            jax.ShapeDtypeStruct((NPAD // P1, 128), jnp.float32),
            jax.ShapeDtypeStruct((NPAD // P1, 128), jnp.float32),
            jax.ShapeDtypeStruct((NPAD // P1, 512), jnp.float32),
        ],
    )(h1p, deg8, deg8, e16, e64)


def _tc_d_body(agg_ref, g1p_ref, dinv16_ref, b1p_ref, zp_ref):
    dinv16 = dinv16_ref[...]
    tot = agg_ref[0] + agg_ref[1] + g1p_ref[...]
    o1 = jnp.maximum(tot * dinv16 + b1p_ref[...], 0.0)
    zp_ref[...] = o1 * dinv16


def _tc_d(agg1p, g1p, dinv16p, b1p):
    return pl.pallas_call(
        _tc_d_body,
        grid=(NPAD // _R,),
        in_specs=[
            pl.BlockSpec((NC, _R // P1, 128), lambda i: (0, i, 0)),
            pl.BlockSpec((_R // P1, 128), lambda i: (i, 0)),
            pl.BlockSpec((_R // P1, 128), lambda i: (i, 0)),
            pl.BlockSpec((1, 128), lambda i: (0, 0)),
        ],
        out_specs=pl.BlockSpec((_R // P1, 128), lambda i: (i, 0)),
        out_shape=jax.ShapeDtypeStruct((NPAD // P1, 128), jnp.float32),
    )(agg1p, g1p, dinv16p, b1p)


def _tc_f_body(agg_ref, zp_ref, dinv64_ref, b2p_ref, w2p_ref, wfc8_ref,
               bfc_ref, out_ref):
    totz = agg_ref[0] + agg_ref[1] + zp_ref[...]
    h2 = jnp.dot(totz, w2p_ref[...],
                 preferred_element_type=jnp.float32)
    o2 = jnp.maximum(h2 * dinv64_ref[...] + b2p_ref[...], 0.0)
    y = jnp.dot(o2, wfc8_ref[...], preferred_element_type=jnp.float32)
    out_ref[...] = jax.nn.sigmoid(y + bfc_ref[0, 0])


def _tc_f(agg2p, zp, dinv64q, b2p, w2p, wfc8, bfc):
    return pl.pallas_call(
        _tc_f_body,
        grid=(NPAD // _R,),
        in_specs=[
            pl.BlockSpec((NC, _R // P1, 128), lambda i: (0, i, 0)),
            pl.BlockSpec((_R // P1, 128), lambda i: (i, 0)),
            pl.BlockSpec((_R // P1, 512), lambda i: (i, 0)),
            pl.BlockSpec((1, 512), lambda i: (0, 0)),
            pl.BlockSpec((128, 512), lambda i: (0, 0)),
            pl.BlockSpec((512, 8), lambda i: (0, 0)),
            pl.BlockSpec((1, 1), lambda i: (0, 0), memory_space=pltpu.SMEM),
        ],
        out_specs=pl.BlockSpec((_R // P1, 8), lambda i: (i, 0)),
        out_shape=jax.ShapeDtypeStruct((NPAD // P1, 8), jnp.float32),
    )(agg2p, zp, dinv64q, b2p, w2p, wfc8, bfc)


def _block_diag(w, k):
    a, b = w.shape
    eye = jnp.eye(k, dtype=w.dtype)
    return (eye[:, None, :, None] * w[None, :, None, :]).reshape(k * a, k * b)


@jax.jit
def kernel(x, edge_index, W1, b1, W2, b2, Wfc, bfc):
    x_pad = jnp.zeros((NPAD, D), jnp.float32).at[:N].set(x)
    x_resh = x_pad.reshape(NPAD // P1, P1 * D)
    w1p = _block_diag(W1, P1)
    w2p = _block_diag(W2, P1)
    wfc8 = _block_diag(Wfc, P1)
    b1p = jnp.tile(b1, P1).reshape(1, 128)
    b2p = jnp.tile(b2, P1).reshape(1, 512)

    e16 = (jnp.arange(8)[:, None] ==
           jnp.arange(128)[None, :] // 16).astype(jnp.float32)
    e64 = (jnp.arange(8)[:, None] ==
           jnp.arange(512)[None, :] // 64).astype(jnp.float32)

    deg = _deg_call(edge_index)
    h1p = _tc_mm1(x_resh, w1p)
    deg8 = deg.reshape(NC, NPAD // 8, 8)
    g1p, dinv16p, dinv64q = _tc_scale(h1p, deg8, e16, e64)
    agg1 = _agg_h1(g1p.reshape(NPAD, H1), edge_index)
    agg1p = agg1.reshape(NC, NPAD // P1, 128)
    zp = _tc_d(agg1p, g1p, dinv16p, b1p)
    agg2 = _agg_h1(zp.reshape(NPAD, H1), edge_index)
    agg2p = agg2.reshape(NC, NPAD // P1, 128)
    outp = _tc_f(agg2p, zp, dinv64q, b2p, w2p, wfc8, bfc.reshape(1, 1))
    return outp.reshape(NPAD, 1)[:N]


# speedup vs baseline: 93.0463x; 1.0082x over previous
import functools

import jax
import jax.numpy as jnp
from jax import lax
from jax.experimental import pallas as pl
from jax.experimental.pallas import tpu as pltpu
from jax.experimental.pallas import tpu_sc as plsc

N = 10000
E = 320000
D = 128
H1 = 16
H2 = 64

NC = 2
NS = 16
NW = NC * NS

NPAD = 10240
RPT = NPAD // NS
EB = 128
CN = 78
NBUF = 6
NG = CN // NBUF
NBA = 8
NGA = CN // NBA
WBC = RPT // EB
IDXB = 32
NX = 4


def _zero_rows(ref, nrows, ncols):
    per_row = ncols // 16
    z = jnp.zeros((16,), jnp.float32)

    def body(t, carry):
        ref[t // per_row, pl.ds((t % per_row) * 16, 16)] = z
        return carry

    lax.fori_loop(0, nrows * per_row, body, 0)


def _edge_base(wid):
    return wid * (CN * EB) + jnp.minimum(wid, NX) * EB


def _preload_idx(ei_hbm, idx3, ebase, wid, sem):
    for k0 in range(0, CN, IDXB):
        descs = [
            pltpu.async_copy(ei_hbm.at[:, pl.ds(ebase + i * EB, EB)],
                             idx3.at[i], sem)
            for i in range(k0, min(k0 + IDXB, CN))
        ]
        for d in descs:
            d.wait()

    @pl.when(wid < NX)
    def _():
        pltpu.async_copy(ei_hbm.at[:, pl.ds(ebase + CN * EB, EB)],
                         idx3.at[CN], sem).wait()


def _deg_body(ei_hbm, out_hbm, idx3, ones_v, stage_v, acc,
              sem_i, ss0, ss1, ss2, ss3, ss4, ss5):
    sems = (ss0, ss1, ss2, ss3, ss4, ss5)
    c = lax.axis_index("c")
    s = lax.axis_index("s")
    row0 = s * RPT
    wid = c * NS + s
    ebase = _edge_base(wid)

    _preload_idx(ei_hbm, idx3, ebase, wid, sem_i)

    z = jnp.zeros((16,), jnp.float32)
    o = jnp.ones((16,), jnp.float32)
    for t in range(EB // 16):
        stage_v[pl.ds(t * 16, 16)] = z
        ones_v[pl.ds(t * 16, 16)] = o
    zd = [
        pltpu.async_copy(stage_v, acc.at[pl.ds(row0 + j * EB, EB)], sem_i)
        for j in range(WBC)
    ]
    for d in zd:
        d.wait()
    plsc.subcore_barrier()

    def grp(t, carry):
        for b in range(NBUF):
            i = t * NBUF + b

            @pl.when(t > 0)
            def _():
                pltpu.make_async_copy(ones_v, acc.at[idx3.at[i, 1]],
                                      sems[b]).wait()

            pltpu.async_copy(ones_v, acc.at[idx3.at[i, 1]], sems[b],
                             add=True)
        return carry

    lax.fori_loop(0, NG, grp, 0)
    for b in range(NBUF):
        pltpu.make_async_copy(ones_v, acc.at[idx3.at[b, 1]], sems[b]).wait()

    @pl.when(wid < NX)
    def _():
        pltpu.async_copy(ones_v, acc.at[idx3.at[CN, 1]], sems[0], add=True)
        pltpu.make_async_copy(ones_v, acc.at[idx3.at[CN, 1]], sems[0]).wait()

    plsc.subcore_barrier()

    for j in range(WBC):
        pltpu.sync_copy(acc.at[pl.ds(row0 + j * EB, EB)], stage_v)
        pltpu.sync_copy(stage_v,
                        out_hbm.at[pl.ds(c * NPAD + row0 + j * EB, EB)])


_deg_call = pl.kernel(
    _deg_body,
    out_type=jax.ShapeDtypeStruct((NC * NPAD,), jnp.float32),
    mesh=plsc.VectorSubcoreMesh(core_axis_name="c", subcore_axis_name="s"),
    compiler_params=pltpu.CompilerParams(use_tc_tiling_on_sc=True),
    scratch_types=[
        pltpu.VMEM((CN + 1, 2, EB), jnp.int32),
        pltpu.VMEM((EB,), jnp.float32),
        pltpu.VMEM((EB,), jnp.float32),
        pltpu.VMEM_SHARED((NPAD,), jnp.float32),
        pltpu.SemaphoreType.DMA,
        pltpu.SemaphoreType.DMA,
        pltpu.SemaphoreType.DMA,
        pltpu.SemaphoreType.DMA,
        pltpu.SemaphoreType.DMA,
        pltpu.SemaphoreType.DMA,
        pltpu.SemaphoreType.DMA,
    ],
)


def _make_agg(F):
    def body(g_hbm, ei_hbm, out_hbm,
             idx3, r0, r1, r2, r3, r4, r5, r6, r7, acc,
             sem_i, sg0, sg1, sg2, sg3, sg4, sg5, sg6, sg7,
             ss0, ss1, ss2, ss3, ss4, ss5, ss6, ss7):
        rows = (r0, r1, r2, r3, r4, r5, r6, r7)
        semg = (sg0, sg1, sg2, sg3, sg4, sg5, sg6, sg7)
        sems = (ss0, ss1, ss2, ss3, ss4, ss5, ss6, ss7)
        c = lax.axis_index("c")
        s = lax.axis_index("s")
        row0 = s * RPT
        wid = c * NS + s
        ebase = _edge_base(wid)

        _preload_idx(ei_hbm, idx3, ebase, wid, sem_i)

        _zero_rows(rows[0], EB, F)
        zd = [
            pltpu.async_copy(rows[0], acc.at[pl.ds(row0 + j * EB, EB)], sem_i)
            for j in range(WBC)
        ]
        for d in zd:
            d.wait()
        plsc.subcore_barrier()

        def grp(t, carry):
            for b in range(NBA):
                i = t * NBA + b

                @pl.when(t > 0)
                def _():
                    pltpu.make_async_copy(rows[b], acc.at[idx3.at[i, 1]],
                                          sems[b]).wait()

                pltpu.async_copy(g_hbm.at[idx3.at[i, 0]], rows[b], semg[b])
            for b in range(NBA):
                i = t * NBA + b
                pltpu.make_async_copy(g_hbm.at[idx3.at[i, 0]], rows[b],
                                      semg[b]).wait()
                pltpu.async_copy(rows[b], acc.at[idx3.at[i, 1]], sems[b],
                                 add=True)
            return carry

        lax.fori_loop(0, NGA, grp, 0)
        ntail = CN - NGA * NBA
        for b in range(ntail):
            i = NGA * NBA + b
            pltpu.make_async_copy(rows[b], acc.at[idx3.at[i, 1]],
                                  sems[b]).wait()
            pltpu.async_copy(g_hbm.at[idx3.at[i, 0]], rows[b], semg[b])
        for b in range(ntail):
            i = NGA * NBA + b
            pltpu.make_async_copy(g_hbm.at[idx3.at[i, 0]], rows[b],
                                  semg[b]).wait()
            pltpu.async_copy(rows[b], acc.at[idx3.at[i, 1]], sems[b],
                             add=True)
        for b in range(NBA):
            pltpu.make_async_copy(rows[b], acc.at[idx3.at[b, 1]],
                                  sems[b]).wait()

        @pl.when(wid < NX)
        def _():
            pltpu.async_copy(g_hbm.at[idx3.at[CN, 0]], rows[0], semg[0])
            pltpu.make_async_copy(g_hbm.at[idx3.at[CN, 0]], rows[0],
                                  semg[0]).wait()
            pltpu.async_copy(rows[0], acc.at[idx3.at[CN, 1]], sems[0],
                             add=True)
            pltpu.make_async_copy(rows[0], acc.at[idx3.at[CN, 1]],
                                  sems[0]).wait()

        plsc.subcore_barrier()

        for j in range(WBC):
            pltpu.async_copy(acc.at[pl.ds(row0 + j * EB, EB)], rows[j],
                             semg[j])
        for j in range(WBC):
            pltpu.make_async_copy(acc.at[pl.ds(row0 + j * EB, EB)], rows[j],
                                  semg[j]).wait()
            pltpu.async_copy(rows[j],
                             out_hbm.at[c, pl.ds(row0 + j * EB, EB)], sems[j])
        for j in range(WBC):
            pltpu.make_async_copy(
                rows[j], out_hbm.at[c, pl.ds(row0, EB)], sems[j]).wait()

    return pl.kernel(
        body,
        out_type=jax.ShapeDtypeStruct((NC, NPAD, F), jnp.float32),
        mesh=plsc.VectorSubcoreMesh(core_axis_name="c", subcore_axis_name="s"),
        compiler_params=pltpu.CompilerParams(use_tc_tiling_on_sc=False),
        scratch_types=[
            pltpu.VMEM((CN + 1, 2, EB), jnp.int32),
            pltpu.VMEM((EB, F), jnp.float32),
            pltpu.VMEM((EB, F), jnp.float32),
            pltpu.VMEM((EB, F), jnp.float32),
            pltpu.VMEM((EB, F), jnp.float32),
            pltpu.VMEM((EB, F), jnp.float32),
            pltpu.VMEM((EB, F), jnp.float32),
            pltpu.VMEM((EB, F), jnp.float32),
            pltpu.VMEM((EB, F), jnp.float32),
            pltpu.VMEM_SHARED((NPAD, F), jnp.float32),
        ] + [pltpu.SemaphoreType.DMA] * 17,
    )


_agg_h1 = _make_agg(H1)


_R = 2048
P1 = 128 // H1
P2 = 128 // H2


def _tc_mm1_body(x_ref, w1p_ref, h1p_ref):
    h1p_ref[...] = jnp.dot(x_ref[...], w1p_ref[...],
                           preferred_element_type=jnp.float32)


def _tc_mm1(x_resh, w1p):
    rb = N // P1
    return pl.pallas_call(
        _tc_mm1_body,
        grid=(1,),
        in_specs=[
            pl.BlockSpec((rb, P1 * D), lambda i: (0, 0)),
            pl.BlockSpec((P1 * D, 128), lambda i: (0, 0)),
        ],
        out_specs=pl.BlockSpec((rb, 128), lambda i: (0, 0)),
        out_shape=jax.ShapeDtypeStruct((N // P1, 128), jnp.float32),
    )(x_resh, w1p)


def _tc_scale_body(h1p_ref, dega_ref, degb_ref, e16_ref,
                   g1p_ref, dinv16_ref, dinv8_ref):
    deg = dega_ref[0] + degb_ref[0] + 1.0
    dinv8 = lax.rsqrt(deg)
    dinv16 = jnp.dot(dinv8, e16_ref[...],
                     preferred_element_type=jnp.float32)
    g1p_ref[...] = h1p_ref[...] * dinv16
    dinv16_ref[...] = dinv16
    dinv8_ref[...] = dinv8


def _tc_scale(h1p, deg8, e16):
    return pl.pallas_call(
        _tc_scale_body,
        grid=(NPAD // _R,),
        in_specs=[
            pl.BlockSpec((_R // P1, 128), lambda i: (i, 0)),
            pl.BlockSpec((1, _R // 8, 8), lambda i: (0, i, 0)),
            pl.BlockSpec((1, _R // 8, 8), lambda i: (1, i, 0)),
            pl.BlockSpec((8, 128), lambda i: (0, 0)),
        ],
        out_specs=[
            pl.BlockSpec((_R // P1, 128), lambda i: (i, 0)),
            pl.BlockSpec((_R // P1, 128), lambda i: (i, 0)),
            pl.BlockSpec((_R // P1, 8), lambda i: (i, 0)),
        ],
        out_shape=[
            jax.ShapeDtypeStruct((NPAD // P1, 128), jnp.float32),
            jax.ShapeDtypeStruct((NPAD // P1, 128), jnp.float32),
            jax.ShapeDtypeStruct((NPAD // P1, 8), jnp.float32),
        ],
    )(h1p, deg8, deg8, e16)


def _tc_d_body(agg_ref, g1p_ref, dinv16_ref, b1p_ref, zp_ref):
    dinv16 = dinv16_ref[...]
    tot = agg_ref[0] + agg_ref[1] + g1p_ref[...]
    o1 = jnp.maximum(tot * dinv16 + b1p_ref[...], 0.0)
    zp_ref[...] = o1 * dinv16


def _tc_d(agg1p, g1p, dinv16p, b1p):
    return pl.pallas_call(
        _tc_d_body,
        grid=(NPAD // _R,),
        in_specs=[
            pl.BlockSpec((NC, _R // P1, 128), lambda i: (0, i, 0)),
            pl.BlockSpec((_R // P1, 128), lambda i: (i, 0)),
            pl.BlockSpec((_R // P1, 128), lambda i: (i, 0)),
            pl.BlockSpec((1, 128), lambda i: (0, 0)),
        ],
        out_specs=pl.BlockSpec((_R // P1, 128), lambda i: (i, 0)),
        out_shape=jax.ShapeDtypeStruct((NPAD // P1, 128), jnp.float32),
    )(agg1p, g1p, dinv16p, b1p)


def _tc_f_body(agg_ref, zp_ref, dinv8_ref, e64_ref, b2p_ref, w2p_ref,
               wfc8_ref, bfc_ref, out_ref):
    totz = agg_ref[0] + agg_ref[1] + zp_ref[...]
    h2 = jnp.dot(totz, w2p_ref[...],
                 preferred_element_type=jnp.float32)
    dinv64 = jnp.dot(dinv8_ref[...], e64_ref[...],
                     preferred_element_type=jnp.float32)
    o2 = jnp.maximum(h2 * dinv64 + b2p_ref[...], 0.0)
    y = jnp.dot(o2, wfc8_ref[...], preferred_element_type=jnp.float32)
    out_ref[...] = jax.nn.sigmoid(y + bfc_ref[0, 0])


def _tc_f(agg2p, zp, dinv8p, e64, b2p, w2p, wfc8, bfc):
    return pl.pallas_call(
        _tc_f_body,
        grid=(NPAD // _R,),
        in_specs=[
            pl.BlockSpec((NC, _R // P1, 128), lambda i: (0, i, 0)),
            pl.BlockSpec((_R // P1, 128), lambda i: (i, 0)),
            pl.BlockSpec((_R // P1, 8), lambda i: (i, 0)),
            pl.BlockSpec((8, 512), lambda i: (0, 0)),
            pl.BlockSpec((1, 512), lambda i: (0, 0)),
            pl.BlockSpec((128, 512), lambda i: (0, 0)),
            pl.BlockSpec((512, 8), lambda i: (0, 0)),
            pl.BlockSpec((1, 1), lambda i: (0, 0), memory_space=pltpu.SMEM),
        ],
        out_specs=pl.BlockSpec((_R // P1, 8), lambda i: (i, 0)),
        out_shape=jax.ShapeDtypeStruct((NPAD // P1, 8), jnp.float32),
    )(agg2p, zp, dinv8p, e64, b2p, w2p, wfc8, bfc)


def _block_diag(w, k):
    a, b = w.shape
    eye = jnp.eye(k, dtype=w.dtype)
    return (eye[:, None, :, None] * w[None, :, None, :]).reshape(k * a, k * b)


@jax.jit
def kernel(x, edge_index, W1, b1, W2, b2, Wfc, bfc):
    x_resh = x.reshape(N // P1, P1 * D)
    w1p = _block_diag(W1, P1)
    w2p = _block_diag(W2, P1)
    wfc8 = _block_diag(Wfc, P1)
    b1p = jnp.tile(b1, P1).reshape(1, 128)
    b2p = jnp.tile(b2, P1).reshape(1, 512)

    e16 = (jnp.arange(8)[:, None] ==
           jnp.arange(128)[None, :] // 16).astype(jnp.float32)
    e64 = (jnp.arange(8)[:, None] ==
           jnp.arange(512)[None, :] // 64).astype(jnp.float32)

    deg = _deg_call(edge_index)
    h1r = _tc_mm1(x_resh, w1p)
    h1p = jnp.zeros((NPAD // P1, 128), jnp.float32).at[:N // P1].set(h1r)
    deg8 = deg.reshape(NC, NPAD // 8, 8)
    g1p, dinv16p, dinv8p = _tc_scale(h1p, deg8, e16)
    agg1 = _agg_h1(g1p.reshape(NPAD, H1), edge_index)
    agg1p = agg1.reshape(NC, NPAD // P1, 128)
    zp = _tc_d(agg1p, g1p, dinv16p, b1p)
    agg2 = _agg_h1(zp.reshape(NPAD, H1), edge_index)
    agg2p = agg2.reshape(NC, NPAD // P1, 128)
    outp = _tc_f(agg2p, zp, dinv8p, e64, b2p, w2p, wfc8,
                 bfc.reshape(1, 1))
    return outp.reshape(NPAD, 1)[:N]


# speedup vs baseline: 98.8561x; 1.0624x over previous
import functools

import jax
import jax.numpy as jnp
from jax import lax
from jax.experimental import pallas as pl
from jax.experimental.pallas import tpu as pltpu
from jax.experimental.pallas import tpu_sc as plsc

N = 10000
E = 320000
D = 128
H1 = 16
H2 = 64

NC = 2
NS = 16
NW = NC * NS

NPAD = 10240
RPT = NPAD // NS
EB = 128
CN = 78
NBUF = 6
NG = CN // NBUF
NBA = 8
NGA = CN // NBA
WBC = RPT // EB
IDXB = 32
NX = 4


def _zero_rows(ref, nrows, ncols):
    per_row = ncols // 16
    z = jnp.zeros((16,), jnp.float32)

    def body(t, carry):
        ref[t // per_row, pl.ds((t % per_row) * 16, 16)] = z
        return carry

    lax.fori_loop(0, nrows * per_row, body, 0)


def _edge_base(wid):
    return wid * (CN * EB) + jnp.minimum(wid, NX) * EB


def _preload_idx(ei_hbm, idx3, ebase, wid, sem):
    for k0 in range(0, CN, IDXB):
        descs = [
            pltpu.async_copy(ei_hbm.at[:, pl.ds(ebase + i * EB, EB)],
                             idx3.at[i], sem)
            for i in range(k0, min(k0 + IDXB, CN))
        ]
        for d in descs:
            d.wait()

    @pl.when(wid < NX)
    def _():
        pltpu.async_copy(ei_hbm.at[:, pl.ds(ebase + CN * EB, EB)],
                         idx3.at[CN], sem).wait()


def _deg_body(ei_hbm, out_hbm, idx3, ones_v, stage_v, acc,
              sem_i, ss0, ss1, ss2, ss3, ss4, ss5):
    sems = (ss0, ss1, ss2, ss3, ss4, ss5)
    c = lax.axis_index("c")
    s = lax.axis_index("s")
    row0 = s * RPT
    wid = c * NS + s
    ebase = _edge_base(wid)

    _preload_idx(ei_hbm, idx3, ebase, wid, sem_i)

    z = jnp.zeros((16,), jnp.float32)
    o = jnp.ones((16,), jnp.float32)
    for t in range(EB // 16):
        stage_v[pl.ds(t * 16, 16)] = z
        ones_v[pl.ds(t * 16, 16)] = o
    zd = [
        pltpu.async_copy(stage_v, acc.at[pl.ds(row0 + j * EB, EB)], sem_i)
        for j in range(WBC)
    ]
    for d in zd:
        d.wait()
    plsc.subcore_barrier()

    def grp(t, carry):
        for b in range(NBUF):
            i = t * NBUF + b

            @pl.when(t > 0)
            def _():
                pltpu.make_async_copy(ones_v, acc.at[idx3.at[i, 1]],
                                      sems[b]).wait()

            pltpu.async_copy(ones_v, acc.at[idx3.at[i, 1]], sems[b],
                             add=True)
        return carry

    lax.fori_loop(0, NG, grp, 0)
    for b in range(NBUF):
        pltpu.make_async_copy(ones_v, acc.at[idx3.at[b, 1]], sems[b]).wait()

    @pl.when(wid < NX)
    def _():
        pltpu.async_copy(ones_v, acc.at[idx3.at[CN, 1]], sems[0], add=True)
        pltpu.make_async_copy(ones_v, acc.at[idx3.at[CN, 1]], sems[0]).wait()

    plsc.subcore_barrier()

    for j in range(WBC):
        pltpu.sync_copy(acc.at[pl.ds(row0 + j * EB, EB)], stage_v)
        pltpu.sync_copy(stage_v,
                        out_hbm.at[pl.ds(c * NPAD + row0 + j * EB, EB)])


_deg_call = pl.kernel(
    _deg_body,
    out_type=jax.ShapeDtypeStruct((NC * NPAD,), jnp.float32),
    mesh=plsc.VectorSubcoreMesh(core_axis_name="c", subcore_axis_name="s"),
    compiler_params=pltpu.CompilerParams(use_tc_tiling_on_sc=True),
    scratch_types=[
        pltpu.VMEM((CN + 1, 2, EB), jnp.int32),
        pltpu.VMEM((EB,), jnp.float32),
        pltpu.VMEM((EB,), jnp.float32),
        pltpu.VMEM_SHARED((NPAD,), jnp.float32),
        pltpu.SemaphoreType.DMA,
        pltpu.SemaphoreType.DMA,
        pltpu.SemaphoreType.DMA,
        pltpu.SemaphoreType.DMA,
        pltpu.SemaphoreType.DMA,
        pltpu.SemaphoreType.DMA,
        pltpu.SemaphoreType.DMA,
    ],
)


def _make_agg(F):
    def body(g_hbm, ei_hbm, out_hbm,
             idx3, r0, r1, r2, r3, r4, r5, r6, r7, acc,
             sem_i, sg0, sg1, sg2, sg3, sg4, sg5, sg6, sg7,
             ss0, ss1, ss2, ss3, ss4, ss5, ss6, ss7):
        rows = (r0, r1, r2, r3, r4, r5, r6, r7)
        semg = (sg0, sg1, sg2, sg3, sg4, sg5, sg6, sg7)
        sems = (ss0, ss1, ss2, ss3, ss4, ss5, ss6, ss7)
        c = lax.axis_index("c")
        s = lax.axis_index("s")
        row0 = s * RPT
        wid = c * NS + s
        ebase = _edge_base(wid)

        _preload_idx(ei_hbm, idx3, ebase, wid, sem_i)

        _zero_rows(rows[0], EB, F)
        zd = [
            pltpu.async_copy(rows[0], acc.at[pl.ds(row0 + j * EB, EB)], sem_i)
            for j in range(WBC)
        ]
        for d in zd:
            d.wait()
        plsc.subcore_barrier()

        def grp(t, carry):
            for b in range(NBA):
                i = t * NBA + b

                @pl.when(t > 0)
                def _():
                    pltpu.make_async_copy(rows[b], acc.at[idx3.at[i, 1]],
                                          sems[b]).wait()

                pltpu.async_copy(g_hbm.at[idx3.at[i, 0]], rows[b], semg[b])
            for b in range(NBA):
                i = t * NBA + b
                pltpu.make_async_copy(g_hbm.at[idx3.at[i, 0]], rows[b],
                                      semg[b]).wait()
                pltpu.async_copy(rows[b], acc.at[idx3.at[i, 1]], sems[b],
                                 add=True)
            return carry

        lax.fori_loop(0, NGA, grp, 0)
        ntail = CN - NGA * NBA
        for b in range(ntail):
            i = NGA * NBA + b
            pltpu.make_async_copy(rows[b], acc.at[idx3.at[i, 1]],
                                  sems[b]).wait()
            pltpu.async_copy(g_hbm.at[idx3.at[i, 0]], rows[b], semg[b])
        for b in range(ntail):
            i = NGA * NBA + b
            pltpu.make_async_copy(g_hbm.at[idx3.at[i, 0]], rows[b],
                                  semg[b]).wait()
            pltpu.async_copy(rows[b], acc.at[idx3.at[i, 1]], sems[b],
                             add=True)
        for b in range(NBA):
            pltpu.make_async_copy(rows[b], acc.at[idx3.at[b, 1]],
                                  sems[b]).wait()

        @pl.when(wid < NX)
        def _():
            pltpu.async_copy(g_hbm.at[idx3.at[CN, 0]], rows[0], semg[0])
            pltpu.make_async_copy(g_hbm.at[idx3.at[CN, 0]], rows[0],
                                  semg[0]).wait()
            pltpu.async_copy(rows[0], acc.at[idx3.at[CN, 1]], sems[0],
                             add=True)
            pltpu.make_async_copy(rows[0], acc.at[idx3.at[CN, 1]],
                                  sems[0]).wait()

        plsc.subcore_barrier()

        for j in range(WBC):
            pltpu.async_copy(acc.at[pl.ds(row0 + j * EB, EB)], rows[j],
                             semg[j])
        for j in range(WBC):
            pltpu.make_async_copy(acc.at[pl.ds(row0 + j * EB, EB)], rows[j],
                                  semg[j]).wait()
            pltpu.async_copy(rows[j],
                             out_hbm.at[c, pl.ds(row0 + j * EB, EB)], sems[j])
        for j in range(WBC):
            pltpu.make_async_copy(
                rows[j], out_hbm.at[c, pl.ds(row0, EB)], sems[j]).wait()

    return pl.kernel(
        body,
        out_type=jax.ShapeDtypeStruct((NC, NPAD, F), jnp.float32),
        mesh=plsc.VectorSubcoreMesh(core_axis_name="c", subcore_axis_name="s"),
        compiler_params=pltpu.CompilerParams(use_tc_tiling_on_sc=False),
        scratch_types=[
            pltpu.VMEM((CN + 1, 2, EB), jnp.int32),
            pltpu.VMEM((EB, F), jnp.float32),
            pltpu.VMEM((EB, F), jnp.float32),
            pltpu.VMEM((EB, F), jnp.float32),
            pltpu.VMEM((EB, F), jnp.float32),
            pltpu.VMEM((EB, F), jnp.float32),
            pltpu.VMEM((EB, F), jnp.float32),
            pltpu.VMEM((EB, F), jnp.float32),
            pltpu.VMEM((EB, F), jnp.float32),
            pltpu.VMEM_SHARED((NPAD, F), jnp.float32),
        ] + [pltpu.SemaphoreType.DMA] * 17,
    )


_agg_h1 = _make_agg(H1)


_R = 2048
P1 = 128 // H1
P2 = 128 // H2


def _tc_ms_body(x_ref, w1p_ref, dega_ref, degb_ref, e16_ref,
                g1p_ref, dinv16_ref, dinv8_ref):
    h1p = jnp.dot(x_ref[...], w1p_ref[...],
                  preferred_element_type=jnp.float32)
    deg = dega_ref[0] + degb_ref[0] + 1.0
    dinv8 = lax.rsqrt(deg)
    dinv16 = jnp.dot(dinv8, e16_ref[...],
                     preferred_element_type=jnp.float32)
    g1p_ref[...] = h1p * dinv16
    dinv16_ref[...] = dinv16
    dinv8_ref[...] = dinv8


def _tc_ms(x_resh, w1p, deg8, e16):
    rb = NPAD // P1
    return pl.pallas_call(
        _tc_ms_body,
        grid=(1,),
        in_specs=[
            pl.BlockSpec((rb, P1 * D), lambda i: (0, 0)),
            pl.BlockSpec((P1 * D, 128), lambda i: (0, 0)),
            pl.BlockSpec((1, rb, 8), lambda i: (0, 0, 0)),
            pl.BlockSpec((1, rb, 8), lambda i: (1, 0, 0)),
            pl.BlockSpec((8, 128), lambda i: (0, 0)),
        ],
        out_specs=[
            pl.BlockSpec((rb, 128), lambda i: (0, 0)),
            pl.BlockSpec((rb, 128), lambda i: (0, 0)),
            pl.BlockSpec((rb, 8), lambda i: (0, 0)),
        ],
        out_shape=[
            jax.ShapeDtypeStruct((rb, 128), jnp.float32),
            jax.ShapeDtypeStruct((rb, 128), jnp.float32),
            jax.ShapeDtypeStruct((rb, 8), jnp.float32),
        ],
    )(x_resh, w1p, deg8, deg8, e16)


def _tc_d_body(agg_ref, g1p_ref, dinv16_ref, b1p_ref, zp_ref):
    dinv16 = dinv16_ref[...]
    tot = agg_ref[0] + agg_ref[1] + g1p_ref[...]
    o1 = jnp.maximum(tot * dinv16 + b1p_ref[...], 0.0)
    zp_ref[...] = o1 * dinv16


def _tc_d(agg1p, g1p, dinv16p, b1p):
    return pl.pallas_call(
        _tc_d_body,
        grid=(NPAD // _R,),
        in_specs=[
            pl.BlockSpec((NC, _R // P1, 128), lambda i: (0, i, 0)),
            pl.BlockSpec((_R // P1, 128), lambda i: (i, 0)),
            pl.BlockSpec((_R // P1, 128), lambda i: (i, 0)),
            pl.BlockSpec((1, 128), lambda i: (0, 0)),
        ],
        out_specs=pl.BlockSpec((_R // P1, 128), lambda i: (i, 0)),
        out_shape=jax.ShapeDtypeStruct((NPAD // P1, 128), jnp.float32),
    )(agg1p, g1p, dinv16p, b1p)


def _tc_f_body(agg_ref, zp_ref, dinv8_ref, e64_ref, b2p_ref, w2p_ref,
               wfc8_ref, bfc_ref, out_ref):
    totz = agg_ref[0] + agg_ref[1] + zp_ref[...]
    h2 = jnp.dot(totz, w2p_ref[...],
                 preferred_element_type=jnp.float32)
    dinv64 = jnp.dot(dinv8_ref[...], e64_ref[...],
                     preferred_element_type=jnp.float32)
    o2 = jnp.maximum(h2 * dinv64 + b2p_ref[...], 0.0)
    y = jnp.dot(o2, wfc8_ref[...], preferred_element_type=jnp.float32)
    out_ref[...] = jax.nn.sigmoid(y + bfc_ref[0, 0])


def _tc_f(agg2p, zp, dinv8p, e64, b2p, w2p, wfc8, bfc):
    return pl.pallas_call(
        _tc_f_body,
        grid=(NPAD // _R,),
        in_specs=[
            pl.BlockSpec((NC, _R // P1, 128), lambda i: (0, i, 0)),
            pl.BlockSpec((_R // P1, 128), lambda i: (i, 0)),
            pl.BlockSpec((_R // P1, 8), lambda i: (i, 0)),
            pl.BlockSpec((8, 512), lambda i: (0, 0)),
            pl.BlockSpec((1, 512), lambda i: (0, 0)),
            pl.BlockSpec((128, 512), lambda i: (0, 0)),
            pl.BlockSpec((512, 8), lambda i: (0, 0)),
            pl.BlockSpec((1, 1), lambda i: (0, 0), memory_space=pltpu.SMEM),
        ],
        out_specs=pl.BlockSpec((_R // P1, 8), lambda i: (i, 0)),
        out_shape=jax.ShapeDtypeStruct((NPAD // P1, 8), jnp.float32),
    )(agg2p, zp, dinv8p, e64, b2p, w2p, wfc8, bfc)


def _block_diag(w, k):
    a, b = w.shape
    eye = jnp.eye(k, dtype=w.dtype)
    return (eye[:, None, :, None] * w[None, :, None, :]).reshape(k * a, k * b)


@jax.jit
def kernel(x, edge_index, W1, b1, W2, b2, Wfc, bfc):
    x_pad = jnp.zeros((NPAD, D), jnp.float32).at[:N].set(x)
    x_resh = x_pad.reshape(NPAD // P1, P1 * D)
    w1p = _block_diag(W1, P1)
    w2p = _block_diag(W2, P1)
    wfc8 = _block_diag(Wfc, P1)
    b1p = jnp.tile(b1, P1).reshape(1, 128)
    b2p = jnp.tile(b2, P1).reshape(1, 512)

    e16 = (jnp.arange(8)[:, None] ==
           jnp.arange(128)[None, :] // 16).astype(jnp.float32)
    e64 = (jnp.arange(8)[:, None] ==
           jnp.arange(512)[None, :] // 64).astype(jnp.float32)

    deg = _deg_call(edge_index)
    deg8 = deg.reshape(NC, NPAD // 8, 8)
    g1p, dinv16p, dinv8p = _tc_ms(x_resh, w1p, deg8, e16)
    agg1 = _agg_h1(g1p.reshape(NPAD, H1), edge_index)
    agg1p = agg1.reshape(NC, NPAD // P1, 128)
    zp = _tc_d(agg1p, g1p, dinv16p, b1p)
    agg2 = _agg_h1(zp.reshape(NPAD, H1), edge_index)
    agg2p = agg2.reshape(NC, NPAD // P1, 128)
    outp = _tc_f(agg2p, zp, dinv8p, e64, b2p, w2p, wfc8,
                 bfc.reshape(1, 1))
    return outp.reshape(NPAD, 1)[:N]


# speedup vs baseline: 99.5419x; 1.0069x over previous
import functools

import jax
import jax.numpy as jnp
from jax import lax
from jax.experimental import pallas as pl
from jax.experimental.pallas import tpu as pltpu
from jax.experimental.pallas import tpu_sc as plsc

N = 10000
E = 320000
D = 128
H1 = 16
H2 = 64

NC = 2
NS = 16
NW = NC * NS

NPAD = 10240
RPT = NPAD // NS
EB = 128
CN = 78
NBUF = 6
NG = CN // NBUF
NBA = 12
NGA = CN // NBA
WBC = RPT // EB
IDXB = 32
NX = 4


def _zero_rows(ref, nrows, ncols):
    per_row = ncols // 16
    z = jnp.zeros((16,), jnp.float32)

    def body(t, carry):
        ref[t // per_row, pl.ds((t % per_row) * 16, 16)] = z
        return carry

    lax.fori_loop(0, nrows * per_row, body, 0)


def _edge_base(wid):
    return wid * (CN * EB) + jnp.minimum(wid, NX) * EB


def _preload_idx(ei_hbm, idx3, ebase, wid, sem):
    for k0 in range(0, CN, IDXB):
        descs = [
            pltpu.async_copy(ei_hbm.at[:, pl.ds(ebase + i * EB, EB)],
                             idx3.at[i], sem)
            for i in range(k0, min(k0 + IDXB, CN))
        ]
        for d in descs:
            d.wait()

    @pl.when(wid < NX)
    def _():
        pltpu.async_copy(ei_hbm.at[:, pl.ds(ebase + CN * EB, EB)],
                         idx3.at[CN], sem).wait()


def _deg_body(ei_hbm, out_hbm, idx3, ones_v, stage_v, acc,
              sem_i, ss0, ss1, ss2, ss3, ss4, ss5):
    sems = (ss0, ss1, ss2, ss3, ss4, ss5)
    c = lax.axis_index("c")
    s = lax.axis_index("s")
    row0 = s * RPT
    wid = c * NS + s
    ebase = _edge_base(wid)

    _preload_idx(ei_hbm, idx3, ebase, wid, sem_i)

    z = jnp.zeros((16,), jnp.float32)
    o = jnp.ones((16,), jnp.float32)
    for t in range(EB // 16):
        stage_v[pl.ds(t * 16, 16)] = z
        ones_v[pl.ds(t * 16, 16)] = o
    zd = [
        pltpu.async_copy(stage_v, acc.at[pl.ds(row0 + j * EB, EB)], sem_i)
        for j in range(WBC)
    ]
    for d in zd:
        d.wait()
    plsc.subcore_barrier()

    def grp(t, carry):
        for b in range(NBUF):
            i = t * NBUF + b

            @pl.when(t > 0)
            def _():
                pltpu.make_async_copy(ones_v, acc.at[idx3.at[i, 1]],
                                      sems[b]).wait()

            pltpu.async_copy(ones_v, acc.at[idx3.at[i, 1]], sems[b],
                             add=True)
        return carry

    lax.fori_loop(0, NG, grp, 0)
    for b in range(NBUF):
        pltpu.make_async_copy(ones_v, acc.at[idx3.at[b, 1]], sems[b]).wait()

    @pl.when(wid < NX)
    def _():
        pltpu.async_copy(ones_v, acc.at[idx3.at[CN, 1]], sems[0], add=True)
        pltpu.make_async_copy(ones_v, acc.at[idx3.at[CN, 1]], sems[0]).wait()

    plsc.subcore_barrier()

    for j in range(WBC):
        pltpu.sync_copy(acc.at[pl.ds(row0 + j * EB, EB)], stage_v)
        pltpu.sync_copy(stage_v,
                        out_hbm.at[pl.ds(c * NPAD + row0 + j * EB, EB)])


_deg_call = pl.kernel(
    _deg_body,
    out_type=jax.ShapeDtypeStruct((NC * NPAD,), jnp.float32),
    mesh=plsc.VectorSubcoreMesh(core_axis_name="c", subcore_axis_name="s"),
    compiler_params=pltpu.CompilerParams(use_tc_tiling_on_sc=True),
    scratch_types=[
        pltpu.VMEM((CN + 1, 2, EB), jnp.int32),
        pltpu.VMEM((EB,), jnp.float32),
        pltpu.VMEM((EB,), jnp.float32),
        pltpu.VMEM_SHARED((NPAD,), jnp.float32),
        pltpu.SemaphoreType.DMA,
        pltpu.SemaphoreType.DMA,
        pltpu.SemaphoreType.DMA,
        pltpu.SemaphoreType.DMA,
        pltpu.SemaphoreType.DMA,
        pltpu.SemaphoreType.DMA,
        pltpu.SemaphoreType.DMA,
    ],
)


def _make_agg(F):
    def body(g_hbm, ei_hbm, out_hbm, idx3, *rest):
        rows = rest[:NBA]
        acc = rest[NBA]
        sem_i = rest[NBA + 1]
        semg = rest[NBA + 2:NBA + 2 + NBA]
        sems = rest[NBA + 2 + NBA:]
        c = lax.axis_index("c")
        s = lax.axis_index("s")
        row0 = s * RPT
        wid = c * NS + s
        ebase = _edge_base(wid)

        _preload_idx(ei_hbm, idx3, ebase, wid, sem_i)

        _zero_rows(rows[0], EB, F)
        zd = [
            pltpu.async_copy(rows[0], acc.at[pl.ds(row0 + j * EB, EB)], sem_i)
            for j in range(WBC)
        ]
        for d in zd:
            d.wait()
        plsc.subcore_barrier()

        def grp(t, carry):
            for b in range(NBA):
                i = t * NBA + b

                @pl.when(t > 0)
                def _():
                    pltpu.make_async_copy(rows[b], acc.at[idx3.at[i, 1]],
                                          sems[b]).wait()

                pltpu.async_copy(g_hbm.at[idx3.at[i, 0]], rows[b], semg[b])
            for b in range(NBA):
                i = t * NBA + b
                pltpu.make_async_copy(g_hbm.at[idx3.at[i, 0]], rows[b],
                                      semg[b]).wait()
                pltpu.async_copy(rows[b], acc.at[idx3.at[i, 1]], sems[b],
                                 add=True)
            return carry

        lax.fori_loop(0, NGA, grp, 0)
        ntail = CN - NGA * NBA
        for b in range(ntail):
            i = NGA * NBA + b
            pltpu.make_async_copy(rows[b], acc.at[idx3.at[i, 1]],
                                  sems[b]).wait()
            pltpu.async_copy(g_hbm.at[idx3.at[i, 0]], rows[b], semg[b])
        for b in range(ntail):
            i = NGA * NBA + b
            pltpu.make_async_copy(g_hbm.at[idx3.at[i, 0]], rows[b],
                                  semg[b]).wait()
            pltpu.async_copy(rows[b], acc.at[idx3.at[i, 1]], sems[b],
                             add=True)
        for b in range(NBA):
            pltpu.make_async_copy(rows[b], acc.at[idx3.at[b, 1]],
                                  sems[b]).wait()

        @pl.when(wid < NX)
        def _():
            pltpu.async_copy(g_hbm.at[idx3.at[CN, 0]], rows[0], semg[0])
            pltpu.make_async_copy(g_hbm.at[idx3.at[CN, 0]], rows[0],
                                  semg[0]).wait()
            pltpu.async_copy(rows[0], acc.at[idx3.at[CN, 1]], sems[0],
                             add=True)
            pltpu.make_async_copy(rows[0], acc.at[idx3.at[CN, 1]],
                                  sems[0]).wait()

        plsc.subcore_barrier()

        for j in range(WBC):
            pltpu.async_copy(acc.at[pl.ds(row0 + j * EB, EB)], rows[j],
                             semg[j])
        for j in range(WBC):
            pltpu.make_async_copy(acc.at[pl.ds(row0 + j * EB, EB)], rows[j],
                                  semg[j]).wait()
            pltpu.async_copy(rows[j],
                             out_hbm.at[c, pl.ds(row0 + j * EB, EB)], sems[j])
        for j in range(WBC):
            pltpu.make_async_copy(
                rows[j], out_hbm.at[c, pl.ds(row0, EB)], sems[j]).wait()

    return pl.kernel(
        body,
        out_type=jax.ShapeDtypeStruct((NC, NPAD, F), jnp.float32),
        mesh=plsc.VectorSubcoreMesh(core_axis_name="c", subcore_axis_name="s"),
        compiler_params=pltpu.CompilerParams(use_tc_tiling_on_sc=False),
        scratch_types=(
            [pltpu.VMEM((CN + 1, 2, EB), jnp.int32)]
            + [pltpu.VMEM((EB, F), jnp.float32)] * NBA
            + [pltpu.VMEM_SHARED((NPAD, F), jnp.float32)]
            + [pltpu.SemaphoreType.DMA] * (2 * NBA + 1)
        ),
    )


_agg_h1 = _make_agg(H1)


_R = 2048
P1 = 128 // H1
P2 = 128 // H2


def _tc_ms_body(x_ref, w1p_ref, dega_ref, degb_ref, e16_ref,
                g1p_ref, dinv16_ref, dinv8_ref):
    h1p = jnp.dot(x_ref[...], w1p_ref[...],
                  preferred_element_type=jnp.float32)
    deg = dega_ref[0] + degb_ref[0] + 1.0
    dinv8 = lax.rsqrt(deg)
    dinv16 = jnp.dot(dinv8, e16_ref[...],
                     preferred_element_type=jnp.float32)
    g1p_ref[...] = h1p * dinv16
    dinv16_ref[...] = dinv16
    dinv8_ref[...] = dinv8


def _tc_ms(x_resh, w1p, deg8, e16):
    rb = NPAD // P1
    return pl.pallas_call(
        _tc_ms_body,
        grid=(1,),
        in_specs=[
            pl.BlockSpec((rb, P1 * D), lambda i: (0, 0)),
            pl.BlockSpec((P1 * D, 128), lambda i: (0, 0)),
            pl.BlockSpec((1, rb, 8), lambda i: (0, 0, 0)),
            pl.BlockSpec((1, rb, 8), lambda i: (1, 0, 0)),
            pl.BlockSpec((8, 128), lambda i: (0, 0)),
        ],
        out_specs=[
            pl.BlockSpec((rb, 128), lambda i: (0, 0)),
            pl.BlockSpec((rb, 128), lambda i: (0, 0)),
            pl.BlockSpec((rb, 8), lambda i: (0, 0)),
        ],
        out_shape=[
            jax.ShapeDtypeStruct((rb, 128), jnp.float32),
            jax.ShapeDtypeStruct((rb, 128), jnp.float32),
            jax.ShapeDtypeStruct((rb, 8), jnp.float32),
        ],
    )(x_resh, w1p, deg8, deg8, e16)


def _tc_d_body(agg_ref, g1p_ref, dinv16_ref, b1p_ref, zp_ref):
    dinv16 = dinv16_ref[...]
    tot = agg_ref[0] + agg_ref[1] + g1p_ref[...]
    o1 = jnp.maximum(tot * dinv16 + b1p_ref[...], 0.0)
    zp_ref[...] = o1 * dinv16


def _tc_d(agg1p, g1p, dinv16p, b1p):
    return pl.pallas_call(
        _tc_d_body,
        grid=(NPAD // _R,),
        in_specs=[
            pl.BlockSpec((NC, _R // P1, 128), lambda i: (0, i, 0)),
            pl.BlockSpec((_R // P1, 128), lambda i: (i, 0)),
            pl.BlockSpec((_R // P1, 128), lambda i: (i, 0)),
            pl.BlockSpec((1, 128), lambda i: (0, 0)),
        ],
        out_specs=pl.BlockSpec((_R // P1, 128), lambda i: (i, 0)),
        out_shape=jax.ShapeDtypeStruct((NPAD // P1, 128), jnp.float32),
    )(agg1p, g1p, dinv16p, b1p)


def _tc_f_body(agg_ref, zp_ref, dinv8_ref, e64_ref, b2p_ref, w2p_ref,
               wfc8_ref, bfc_ref, out_ref):
    totz = agg_ref[0] + agg_ref[1] + zp_ref[...]
    h2 = jnp.dot(totz, w2p_ref[...],
                 preferred_element_type=jnp.float32)
    dinv64 = jnp.dot(dinv8_ref[...], e64_ref[...],
                     preferred_element_type=jnp.float32)
    o2 = jnp.maximum(h2 * dinv64 + b2p_ref[...], 0.0)
    y = jnp.dot(o2, wfc8_ref[...], preferred_element_type=jnp.float32)
    out_ref[...] = jax.nn.sigmoid(y + bfc_ref[0, 0])


def _tc_f(agg2p, zp, dinv8p, e64, b2p, w2p, wfc8, bfc):
    return pl.pallas_call(
        _tc_f_body,
        grid=(NPAD // _R,),
        in_specs=[
            pl.BlockSpec((NC, _R // P1, 128), lambda i: (0, i, 0)),
            pl.BlockSpec((_R // P1, 128), lambda i: (i, 0)),
            pl.BlockSpec((_R // P1, 8), lambda i: (i, 0)),
            pl.BlockSpec((8, 512), lambda i: (0, 0)),
            pl.BlockSpec((1, 512), lambda i: (0, 0)),
            pl.BlockSpec((128, 512), lambda i: (0, 0)),
            pl.BlockSpec((512, 8), lambda i: (0, 0)),
            pl.BlockSpec((1, 1), lambda i: (0, 0), memory_space=pltpu.SMEM),
        ],
        out_specs=pl.BlockSpec((_R // P1, 8), lambda i: (i, 0)),
        out_shape=jax.ShapeDtypeStruct((NPAD // P1, 8), jnp.float32),
    )(agg2p, zp, dinv8p, e64, b2p, w2p, wfc8, bfc)


def _block_diag(w, k):
    a, b = w.shape
    eye = jnp.eye(k, dtype=w.dtype)
    return (eye[:, None, :, None] * w[None, :, None, :]).reshape(k * a, k * b)


@jax.jit
def kernel(x, edge_index, W1, b1, W2, b2, Wfc, bfc):
    x_pad = jnp.zeros((NPAD, D), jnp.float32).at[:N].set(x)
    x_resh = x_pad.reshape(NPAD // P1, P1 * D)
    w1p = _block_diag(W1, P1)
    w2p = _block_diag(W2, P1)
    wfc8 = _block_diag(Wfc, P1)
    b1p = jnp.tile(b1, P1).reshape(1, 128)
    b2p = jnp.tile(b2, P1).reshape(1, 512)

    e16 = (jnp.arange(8)[:, None] ==
           jnp.arange(128)[None, :] // 16).astype(jnp.float32)
    e64 = (jnp.arange(8)[:, None] ==
           jnp.arange(512)[None, :] // 64).astype(jnp.float32)

    deg = _deg_call(edge_index)
    deg8 = deg.reshape(NC, NPAD // 8, 8)
    g1p, dinv16p, dinv8p = _tc_ms(x_resh, w1p, deg8, e16)
    agg1 = _agg_h1(g1p.reshape(NPAD, H1), edge_index)
    agg1p = agg1.reshape(NC, NPAD // P1, 128)
    zp = _tc_d(agg1p, g1p, dinv16p, b1p)
    agg2 = _agg_h1(zp.reshape(NPAD, H1), edge_index)
    agg2p = agg2.reshape(NC, NPAD // P1, 128)
    outp = _tc_f(agg2p, zp, dinv8p, e64, b2p, w2p, wfc8,
                 bfc.reshape(1, 1))
    return outp.reshape(NPAD, 1)[:N]


# speedup vs baseline: 102.7304x; 1.0320x over previous
import functools

import jax
import jax.numpy as jnp
from jax import lax
from jax.experimental import pallas as pl
from jax.experimental.pallas import tpu as pltpu
from jax.experimental.pallas import tpu_sc as plsc

N = 10000
E = 320000
D = 128
H1 = 16
H2 = 64

NC = 2
NS = 16
NW = NC * NS

NPAD = 10240
RPT = NPAD // NS
EB = 128
CN = 78
NBUF = 6
NG = CN // NBUF
NBA = 12
NGA = CN // NBA
WBC = RPT // EB
IDXB = 32
NX = 4


def _zero_rows(ref, nrows, ncols):
    per_row = ncols // 16
    z = jnp.zeros((16,), jnp.float32)

    def body(t, carry):
        ref[t // per_row, pl.ds((t % per_row) * 16, 16)] = z
        return carry

    lax.fori_loop(0, nrows * per_row, body, 0)


def _edge_base(wid):
    return wid * (CN * EB) + jnp.minimum(wid, NX) * EB


def _preload_idx(ei_hbm, idx3, ebase, wid, sem):
    for k0 in range(0, CN, IDXB):
        descs = [
            pltpu.async_copy(ei_hbm.at[:, pl.ds(ebase + i * EB, EB)],
                             idx3.at[i], sem)
            for i in range(k0, min(k0 + IDXB, CN))
        ]
        for d in descs:
            d.wait()

    @pl.when(wid < NX)
    def _():
        pltpu.async_copy(ei_hbm.at[:, pl.ds(ebase + CN * EB, EB)],
                         idx3.at[CN], sem).wait()


def _deg_body(ei_hbm, out_hbm, idx3, ones_v, stage_v, acc,
              sem_i, ss0, ss1, ss2, ss3, ss4, ss5):
    sems = (ss0, ss1, ss2, ss3, ss4, ss5)
    c = lax.axis_index("c")
    s = lax.axis_index("s")
    row0 = s * RPT
    wid = c * NS + s
    ebase = _edge_base(wid)

    _preload_idx(ei_hbm, idx3, ebase, wid, sem_i)

    z = jnp.zeros((16,), jnp.float32)
    o = jnp.ones((16,), jnp.float32)
    for t in range(EB // 16):
        stage_v[pl.ds(t * 16, 16)] = z
        ones_v[pl.ds(t * 16, 16)] = o
    zd = [
        pltpu.async_copy(stage_v, acc.at[pl.ds(row0 + j * EB, EB)], sem_i)
        for j in range(WBC)
    ]
    for d in zd:
        d.wait()
    plsc.subcore_barrier()

    def grp(t, carry):
        for b in range(NBUF):
            i = t * NBUF + b

            @pl.when(t > 0)
            def _():
                pltpu.make_async_copy(ones_v, acc.at[idx3.at[i, 1]],
                                      sems[b]).wait()

            pltpu.async_copy(ones_v, acc.at[idx3.at[i, 1]], sems[b],
                             add=True)
        return carry

    lax.fori_loop(0, NG, grp, 0)
    for b in range(NBUF):
        pltpu.make_async_copy(ones_v, acc.at[idx3.at[b, 1]], sems[b]).wait()

    @pl.when(wid < NX)
    def _():
        pltpu.async_copy(ones_v, acc.at[idx3.at[CN, 1]], sems[0], add=True)
        pltpu.make_async_copy(ones_v, acc.at[idx3.at[CN, 1]], sems[0]).wait()

    plsc.subcore_barrier()

    for j in range(WBC):
        pltpu.sync_copy(acc.at[pl.ds(row0 + j * EB, EB)], stage_v)
        pltpu.sync_copy(stage_v,
                        out_hbm.at[pl.ds(c * NPAD + row0 + j * EB, EB)])


_deg_call = pl.kernel(
    _deg_body,
    out_type=jax.ShapeDtypeStruct((NC * NPAD,), jnp.float32),
    mesh=plsc.VectorSubcoreMesh(core_axis_name="c", subcore_axis_name="s"),
    compiler_params=pltpu.CompilerParams(use_tc_tiling_on_sc=True),
    scratch_types=[
        pltpu.VMEM((CN + 1, 2, EB), jnp.int32),
        pltpu.VMEM((EB,), jnp.float32),
        pltpu.VMEM((EB,), jnp.float32),
        pltpu.VMEM_SHARED((NPAD,), jnp.float32),
        pltpu.SemaphoreType.DMA,
        pltpu.SemaphoreType.DMA,
        pltpu.SemaphoreType.DMA,
        pltpu.SemaphoreType.DMA,
        pltpu.SemaphoreType.DMA,
        pltpu.SemaphoreType.DMA,
        pltpu.SemaphoreType.DMA,
    ],
)


def _make_agg(F):
    def body(g_hbm, ei_hbm, out_hbm, idx3, *rest):
        rows = rest[:NBA]
        acc = rest[NBA]
        sem_i = rest[NBA + 1]
        semg = rest[NBA + 2:NBA + 2 + NBA]
        sems = rest[NBA + 2 + NBA:]
        c = lax.axis_index("c")
        s = lax.axis_index("s")
        row0 = s * RPT
        wid = c * NS + s
        ebase = _edge_base(wid)

        _preload_idx(ei_hbm, idx3, ebase, wid, sem_i)

        _zero_rows(rows[0], EB, F)
        zd = [
            pltpu.async_copy(rows[0], acc.at[pl.ds(row0 + j * EB, EB)], sem_i)
            for j in range(WBC)
        ]
        for d in zd:
            d.wait()
        plsc.subcore_barrier()

        def grp(t, carry):
            for b in range(NBA):
                i = t * NBA + b

                @pl.when(t > 0)
                def _():
                    pltpu.make_async_copy(rows[b], acc.at[idx3.at[i, 1]],
                                          sems[b]).wait()

                pltpu.async_copy(g_hbm.at[idx3.at[i, 0]], rows[b], semg[b])
            for b in range(NBA):
                i = t * NBA + b
                pltpu.make_async_copy(g_hbm.at[idx3.at[i, 0]], rows[b],
                                      semg[b]).wait()
                pltpu.async_copy(rows[b], acc.at[idx3.at[i, 1]], sems[b],
                                 add=True)
            return carry

        lax.fori_loop(0, NGA, grp, 0)
        ntail = CN - NGA * NBA
        for b in range(ntail):
            i = NGA * NBA + b
            pltpu.make_async_copy(rows[b], acc.at[idx3.at[i, 1]],
                                  sems[b]).wait()
            pltpu.async_copy(g_hbm.at[idx3.at[i, 0]], rows[b], semg[b])
        for b in range(ntail):
            i = NGA * NBA + b
            pltpu.make_async_copy(g_hbm.at[idx3.at[i, 0]], rows[b],
                                  semg[b]).wait()
            pltpu.async_copy(rows[b], acc.at[idx3.at[i, 1]], sems[b],
                             add=True)
        for b in range(NBA):
            pltpu.make_async_copy(rows[b], acc.at[idx3.at[b, 1]],
                                  sems[b]).wait()

        @pl.when(wid < NX)
        def _():
            pltpu.async_copy(g_hbm.at[idx3.at[CN, 0]], rows[0], semg[0])
            pltpu.make_async_copy(g_hbm.at[idx3.at[CN, 0]], rows[0],
                                  semg[0]).wait()
            pltpu.async_copy(rows[0], acc.at[idx3.at[CN, 1]], sems[0],
                             add=True)
            pltpu.make_async_copy(rows[0], acc.at[idx3.at[CN, 1]],
                                  sems[0]).wait()

        plsc.subcore_barrier()

        for j in range(WBC):
            pltpu.async_copy(acc.at[pl.ds(row0 + j * EB, EB)], rows[j],
                             semg[j])
        for j in range(WBC):
            pltpu.make_async_copy(acc.at[pl.ds(row0 + j * EB, EB)], rows[j],
                                  semg[j]).wait()
            pltpu.async_copy(rows[j],
                             out_hbm.at[c, pl.ds(row0 + j * EB, EB)], sems[j])
        for j in range(WBC):
            pltpu.make_async_copy(
                rows[j], out_hbm.at[c, pl.ds(row0, EB)], sems[j]).wait()

    return pl.kernel(
        body,
        out_type=jax.ShapeDtypeStruct((NC, NPAD, F), jnp.float32),
        mesh=plsc.VectorSubcoreMesh(core_axis_name="c", subcore_axis_name="s"),
        compiler_params=pltpu.CompilerParams(use_tc_tiling_on_sc=False),
        scratch_types=(
            [pltpu.VMEM((CN + 1, 2, EB), jnp.int32)]
            + [pltpu.VMEM((EB, F), jnp.float32)] * NBA
            + [pltpu.VMEM_SHARED((NPAD, F), jnp.float32)]
            + [pltpu.SemaphoreType.DMA] * (2 * NBA + 1)
        ),
    )


_agg_h1 = _make_agg(H1)


_R = 2048
P1 = 128 // H1
P2 = 128 // H2


def _tc_ms_body(x_ref, w1p_ref, dega_ref, degb_ref, e16_ref,
                g1p_ref, dinv16_ref, dinv8_ref):
    h1p = jnp.dot(x_ref[...], w1p_ref[...],
                  preferred_element_type=jnp.float32)
    deg = dega_ref[0] + degb_ref[0] + 1.0
    dinv8 = lax.rsqrt(deg)
    dinv16 = jnp.dot(dinv8, e16_ref[...],
                     preferred_element_type=jnp.float32)
    g1p_ref[...] = h1p * dinv16
    dinv16_ref[...] = dinv16
    dinv8_ref[...] = dinv8


def _tc_ms(x_resh, w1p, deg8, e16):
    rb = NPAD // P1
    return pl.pallas_call(
        _tc_ms_body,
        grid=(1,),
        in_specs=[
            pl.BlockSpec((rb, P1 * D), lambda i: (0, 0)),
            pl.BlockSpec((P1 * D, 128), lambda i: (0, 0)),
            pl.BlockSpec((1, rb, 8), lambda i: (0, 0, 0)),
            pl.BlockSpec((1, rb, 8), lambda i: (1, 0, 0)),
            pl.BlockSpec((8, 128), lambda i: (0, 0)),
        ],
        out_specs=[
            pl.BlockSpec((rb, 128), lambda i: (0, 0)),
            pl.BlockSpec((rb, 128), lambda i: (0, 0)),
            pl.BlockSpec((rb, 8), lambda i: (0, 0)),
        ],
        out_shape=[
            jax.ShapeDtypeStruct((rb, 128), jnp.float32),
            jax.ShapeDtypeStruct((rb, 128), jnp.float32),
            jax.ShapeDtypeStruct((rb, 8), jnp.float32),
        ],
    )(x_resh, w1p, deg8, deg8, e16)


def _tc_d_body(agg_ref, g1p_ref, dinv16_ref, b1p_ref, zp_ref):
    dinv16 = dinv16_ref[...]
    tot = agg_ref[0] + agg_ref[1] + g1p_ref[...]
    o1 = jnp.maximum(tot * dinv16 + b1p_ref[...], 0.0)
    zp_ref[...] = o1 * dinv16


def _tc_d(agg1p, g1p, dinv16p, b1p):
    rb = NPAD // P1
    return pl.pallas_call(
        _tc_d_body,
        grid=(1,),
        in_specs=[
            pl.BlockSpec((NC, rb, 128), lambda i: (0, 0, 0)),
            pl.BlockSpec((rb, 128), lambda i: (0, 0)),
            pl.BlockSpec((rb, 128), lambda i: (0, 0)),
            pl.BlockSpec((1, 128), lambda i: (0, 0)),
        ],
        out_specs=pl.BlockSpec((rb, 128), lambda i: (0, 0)),
        out_shape=jax.ShapeDtypeStruct((rb, 128), jnp.float32),
    )(agg1p, g1p, dinv16p, b1p)


def _tc_f_body(agg_ref, zp_ref, dinv8_ref, e64_ref, b2p_ref, w2p_ref,
               wfc8_ref, bfc_ref, out_ref):
    totz = agg_ref[0] + agg_ref[1] + zp_ref[...]
    h2 = jnp.dot(totz, w2p_ref[...],
                 preferred_element_type=jnp.float32)
    dinv64 = jnp.dot(dinv8_ref[...], e64_ref[...],
                     preferred_element_type=jnp.float32)
    o2 = jnp.maximum(h2 * dinv64 + b2p_ref[...], 0.0)
    y = jnp.dot(o2, wfc8_ref[...], preferred_element_type=jnp.float32)
    out_ref[...] = jax.nn.sigmoid(y + bfc_ref[0, 0])


def _tc_f(agg2p, zp, dinv8p, e64, b2p, w2p, wfc8, bfc):
    rb = NPAD // P1
    return pl.pallas_call(
        _tc_f_body,
        grid=(1,),
        in_specs=[
            pl.BlockSpec((NC, rb, 128), lambda i: (0, 0, 0)),
            pl.BlockSpec((rb, 128), lambda i: (0, 0)),
            pl.BlockSpec((rb, 8), lambda i: (0, 0)),
            pl.BlockSpec((8, 512), lambda i: (0, 0)),
            pl.BlockSpec((1, 512), lambda i: (0, 0)),
            pl.BlockSpec((128, 512), lambda i: (0, 0)),
            pl.BlockSpec((512, 8), lambda i: (0, 0)),
            pl.BlockSpec((1, 1), lambda i: (0, 0), memory_space=pltpu.SMEM),
        ],
        out_specs=pl.BlockSpec((rb, 8), lambda i: (0, 0)),
        out_shape=jax.ShapeDtypeStruct((rb, 8), jnp.float32),
    )(agg2p, zp, dinv8p, e64, b2p, w2p, wfc8, bfc)


def _block_diag(w, k):
    a, b = w.shape
    eye = jnp.eye(k, dtype=w.dtype)
    return (eye[:, None, :, None] * w[None, :, None, :]).reshape(k * a, k * b)


@jax.jit
def kernel(x, edge_index, W1, b1, W2, b2, Wfc, bfc):
    x_pad = jnp.zeros((NPAD, D), jnp.float32).at[:N].set(x)
    x_resh = x_pad.reshape(NPAD // P1, P1 * D)
    w1p = _block_diag(W1, P1)
    w2p = _block_diag(W2, P1)
    wfc8 = _block_diag(Wfc, P1)
    b1p = jnp.tile(b1, P1).reshape(1, 128)
    b2p = jnp.tile(b2, P1).reshape(1, 512)

    e16 = (jnp.arange(8)[:, None] ==
           jnp.arange(128)[None, :] // 16).astype(jnp.float32)
    e64 = (jnp.arange(8)[:, None] ==
           jnp.arange(512)[None, :] // 64).astype(jnp.float32)

    deg = _deg_call(edge_index)
    deg8 = deg.reshape(NC, NPAD // 8, 8)
    g1p, dinv16p, dinv8p = _tc_ms(x_resh, w1p, deg8, e16)
    agg1 = _agg_h1(g1p.reshape(NPAD, H1), edge_index)
    agg1p = agg1.reshape(NC, NPAD // P1, 128)
    zp = _tc_d(agg1p, g1p, dinv16p, b1p)
    agg2 = _agg_h1(zp.reshape(NPAD, H1), edge_index)
    agg2p = agg2.reshape(NC, NPAD // P1, 128)
    outp = _tc_f(agg2p, zp, dinv8p, e64, b2p, w2p, wfc8,
                 bfc.reshape(1, 1))
    return outp.reshape(NPAD, 1)[:N]
